# baseline (speedup 1.0000x reference)
import jax
import jax.numpy as jnp
from jax import lax
from jax.experimental import pallas as pl
from jax.experimental.pallas import tpu as pltpu
from jax.experimental.pallas import tpu_sc as plsc

NN = 10000
NE = 160000
DIN = 128
H = 16
EHID = 64
NSTEPS = 3
ALPHA = 0.5
BETA = 1.0 / NSTEPS

NC = 2
NS = 16
NW = NC * NS
EPW = NE // NW
NPT = NN // NS

_SC_MESH = plsc.VectorSubcoreMesh(core_axis_name="c", subcore_axis_name="s")


def _sc_gather_body(table_hbm, idx_hbm, out_hbm, idx_v, rows_v, sem):
    wid = lax.axis_index("s") * NC + lax.axis_index("c")
    base = wid * EPW
    pltpu.sync_copy(idx_hbm.at[pl.ds(base, EPW)], idx_v)
    pltpu.async_copy(table_hbm.at[idx_v], rows_v, sem).wait()
    pltpu.sync_copy(rows_v, out_hbm.at[pl.ds(base, EPW)])


_sc_gather = pl.kernel(
    _sc_gather_body,
    out_type=jax.ShapeDtypeStruct((NE, H), jnp.float32),
    mesh=_SC_MESH,
    scratch_types=[
        pltpu.VMEM((EPW,), jnp.int32),
        pltpu.VMEM((EPW, H), jnp.float32),
        pltpu.SemaphoreType.DMA,
    ],
    compiler_params=pltpu.CompilerParams(use_tc_tiling_on_sc=False),
)


def _sc_gather2_body(ta_hbm, tb_hbm, ia_hbm, ib_hbm, oa_hbm, ob_hbm,
                     idx_v, rows_v, sem):
    wid = lax.axis_index("s") * NC + lax.axis_index("c")
    base = wid * EPW
    pltpu.sync_copy(ia_hbm.at[pl.ds(base, EPW)], idx_v)
    pltpu.async_copy(ta_hbm.at[idx_v], rows_v, sem).wait()
    pltpu.sync_copy(rows_v, oa_hbm.at[pl.ds(base, EPW)])
    pltpu.sync_copy(ib_hbm.at[pl.ds(base, EPW)], idx_v)
    pltpu.async_copy(tb_hbm.at[idx_v], rows_v, sem).wait()
    pltpu.sync_copy(rows_v, ob_hbm.at[pl.ds(base, EPW)])


_sc_gather2 = pl.kernel(
    _sc_gather2_body,
    out_type=(jax.ShapeDtypeStruct((NE, H), jnp.float32),
              jax.ShapeDtypeStruct((NE, H), jnp.float32)),
    mesh=_SC_MESH,
    scratch_types=[
        pltpu.VMEM((EPW,), jnp.int32),
        pltpu.VMEM((EPW, H), jnp.float32),
        pltpu.SemaphoreType.DMA,
    ],
    compiler_params=pltpu.CompilerParams(use_tc_tiling_on_sc=False),
)


def _sc_scatter_body(msg_hbm, dst_hbm, out_hbm, idx_v, msg_v, wb_v, acc_sh):
    cid = lax.axis_index("c")
    sid = lax.axis_index("s")
    wid = sid * NC + cid
    base = wid * EPW

    def zrow(i, carry):
        wb_v[i, :] = jnp.zeros((H,), jnp.float32)
        return carry

    lax.fori_loop(0, NPT, zrow, 0)
    pltpu.sync_copy(wb_v, acc_sh.at[pl.ds(sid * NPT, NPT)])
    plsc.subcore_barrier()
    pltpu.sync_copy(dst_hbm.at[pl.ds(base, EPW)], idx_v)
    pltpu.sync_copy(msg_hbm.at[pl.ds(base, EPW)], msg_v)
    pltpu.sync_copy(msg_v, acc_sh.at[idx_v], add=True)
    plsc.subcore_barrier()
    pltpu.sync_copy(acc_sh.at[pl.ds(sid * NPT, NPT)], wb_v)
    pltpu.sync_copy(wb_v, out_hbm.at[cid, pl.ds(sid * NPT, NPT)])


_sc_scatter = pl.kernel(
    _sc_scatter_body,
    out_type=jax.ShapeDtypeStruct((NC, NN, H), jnp.float32),
    mesh=_SC_MESH,
    scratch_types=[
        pltpu.VMEM((EPW,), jnp.int32),
        pltpu.VMEM((EPW, H), jnp.float32),
        pltpu.VMEM((NPT, H), jnp.float32),
        pltpu.VMEM_SHARED((NN, H), jnp.float32),
    ],
    compiler_params=pltpu.CompilerParams(use_tc_tiling_on_sc=False),
)


_CHK = 1000
_EPS = NE // NS
_NB_S = _EPS // _CHK
_NB_G = EPW // _CHK


def _sc_sug_body(msg_hbm, dst_hbm, src_hbm, old_hbm, h0_hbm, w1_hbm, bv_hbm,
                 outn_hbm, xs_hbm,
                 idx_v, buf_v, agg_v, old_v, h0_v, new_v, w1_v, bv_v,
                 acc_sh, tab_sh):
    cid = lax.axis_index("c")
    sid = lax.axis_index("s")
    wid = sid * NC + cid
    nbase = sid * NPT

    def zrow(i, c):
        new_v[i, :] = jnp.zeros((H,), jnp.float32)
        return c

    lax.fori_loop(0, NPT, zrow, 0)
    pltpu.sync_copy(new_v, acc_sh.at[pl.ds(nbase, NPT)])
    pltpu.sync_copy(w1_hbm, w1_v)
    pltpu.sync_copy(bv_hbm, bv_v)
    plsc.subcore_barrier()
    sbase = sid * _EPS
    for b in range(_NB_S):
        off = sbase + b * _CHK
        pltpu.sync_copy(dst_hbm.at[pl.ds(off, _CHK)], idx_v)
        pltpu.sync_copy(msg_hbm.at[pl.ds(off, _CHK)], buf_v)
        pltpu.sync_copy(buf_v, acc_sh.at[idx_v], add=True)
    plsc.subcore_barrier()
    pltpu.sync_copy(acc_sh.at[pl.ds(nbase, NPT)], agg_v)
    pltpu.sync_copy(old_hbm.at[pl.ds(nbase, NPT)], old_v)
    pltpu.sync_copy(h0_hbm.at[pl.ds(nbase, NPT)], h0_v)

    def urow(r, c):
        rst = agg_v[r, :] + old_v[r, :] + bv_v[0, :]
        temp = ALPHA * rst + (1.0 - ALPHA) * h0_v[r, :]
        z = BETA * bv_v[1, :] + (1.0 - BETA) * temp
        for i in range(H):
            z = z + (BETA * temp[i]) * w1_v[i, :]
        new_v[r, :] = jnp.maximum(z, 0.0)
        return c

    lax.fori_loop(0, NPT, urow, 0)
    pltpu.sync_copy(new_v, tab_sh.at[pl.ds(nbase, NPT)])

    @pl.when(cid == 0)
    def _():
        pltpu.sync_copy(new_v, outn_hbm.at[pl.ds(nbase, NPT)])

    plsc.subcore_barrier()
    gbase = wid * EPW
    for b in range(_NB_G):
        off = gbase + b * _CHK
        pltpu.sync_copy(src_hbm.at[pl.ds(off, _CHK)], idx_v)
        pltpu.sync_copy(tab_sh.at[idx_v], buf_v)
        pltpu.sync_copy(buf_v, xs_hbm.at[pl.ds(off, _CHK)])


_sc_sug = pl.kernel(
    _sc_sug_body,
    out_type=(jax.ShapeDtypeStruct((NN, H), jnp.float32),
              jax.ShapeDtypeStruct((NE, H), jnp.float32)),
    mesh=_SC_MESH,
    scratch_types=[
        pltpu.VMEM((_CHK,), jnp.int32),
        pltpu.VMEM((_CHK, H), jnp.float32),
        pltpu.VMEM((NPT, H), jnp.float32),
        pltpu.VMEM((NPT, H), jnp.float32),
        pltpu.VMEM((NPT, H), jnp.float32),
        pltpu.VMEM((NPT, H), jnp.float32),
        pltpu.VMEM((H, H), jnp.float32),
        pltpu.VMEM((2, H), jnp.float32),
        pltpu.VMEM_SHARED((NN, H), jnp.float32),
        pltpu.VMEM_SHARED((NN, H), jnp.float32),
    ],
    compiler_params=pltpu.CompilerParams(use_tc_tiling_on_sc=False),
)


_NT = 2000
_ET = 8000
_EH = 16000


def _lin0_body(x_ref, w_ref, b_ref, o_ref):
    o_ref[...] = jnp.maximum(x_ref[...] @ w_ref[...] + b_ref[...], 0.0)


_lin0 = pl.pallas_call(
    _lin0_body,
    grid=(NN // _NT,),
    in_specs=[pl.BlockSpec((_NT, DIN), lambda i: (i, 0)),
              pl.BlockSpec((DIN, H), lambda i: (0, 0)),
              pl.BlockSpec((1, H), lambda i: (0, 0))],
    out_specs=pl.BlockSpec((_NT, H), lambda i: (i, 0)),
    out_shape=jax.ShapeDtypeStruct((NN, H), jnp.float32),
)


def _msg_body(xs_ref, ef_ref, we1_ref, be1_ref, we2_ref, be2_ref, r_ref,
              o_ref):
    u = jnp.maximum(ef_ref[...] @ we1_ref[...] + be1_ref[...], 0.0)
    ew = u @ we2_ref[...] + be2_ref[...]
    p = (xs_ref[...] @ r_ref[...]) * ew
    p = p[:, :128] + p[:, 128:]
    p = p[:, :64] + p[:, 64:]
    p = p[:, :32] + p[:, 32:]
    o_ref[...] = p[:, :16] + p[:, 16:]


_msg = pl.pallas_call(
    _msg_body,
    grid=(NE // _ET,),
    in_specs=[pl.BlockSpec((_ET, H), lambda i: (i, 0)),
              pl.BlockSpec((_ET, H), lambda i: (i, 0)),
              pl.BlockSpec((H, EHID), lambda i: (0, 0)),
              pl.BlockSpec((1, EHID), lambda i: (0, 0)),
              pl.BlockSpec((EHID, H * H), lambda i: (0, 0)),
              pl.BlockSpec((1, H * H), lambda i: (0, 0)),
              pl.BlockSpec((H, H * H), lambda i: (0, 0))],
    out_specs=pl.BlockSpec((_ET, H), lambda i: (i, 0)),
    out_shape=jax.ShapeDtypeStruct((NE, H), jnp.float32),
)


def _upd_body(a0_ref, a1_ref, out_ref, h0_ref, bc_ref, w1_ref, b1_ref, o_ref):
    rst = a0_ref[...] + a1_ref[...] + out_ref[...] + bc_ref[...]
    temp = ALPHA * rst + (1.0 - ALPHA) * h0_ref[...]
    z = temp @ w1_ref[...] + b1_ref[...]
    o_ref[...] = jnp.maximum(BETA * z + (1.0 - BETA) * temp, 0.0)


_upd = pl.pallas_call(
    _upd_body,
    grid=(NN // _NT,),
    in_specs=[pl.BlockSpec((_NT, H), lambda i: (i, 0)),
              pl.BlockSpec((_NT, H), lambda i: (i, 0)),
              pl.BlockSpec((_NT, H), lambda i: (i, 0)),
              pl.BlockSpec((_NT, H), lambda i: (i, 0)),
              pl.BlockSpec((1, H), lambda i: (0, 0)),
              pl.BlockSpec((H, H), lambda i: (0, 0)),
              pl.BlockSpec((1, H), lambda i: (0, 0))],
    out_specs=pl.BlockSpec((_NT, H), lambda i: (i, 0)),
    out_shape=jax.ShapeDtypeStruct((NN, H), jnp.float32),
)


def _head_body(out_ref, mean_ref, var_ref, gam_ref, bet_ref, w2r_ref, w3_ref,
               b3_ref, ybn_ref, yw_ref, ysig_ref):
    ybn = ((out_ref[...] - mean_ref[...]) * lax.rsqrt(var_ref[...] + 1e-5)
           * gam_ref[...] + bet_ref[...])
    ybn_ref[...] = ybn
    yw_ref[...] = ybn * w2r_ref[...]
    ysig_ref[...] = jax.nn.sigmoid(ybn @ w3_ref[...] + b3_ref[...])


_head = pl.pallas_call(
    _head_body,
    grid=(NN // _NT,),
    in_specs=[pl.BlockSpec((_NT, H), lambda i: (i, 0)),
              pl.BlockSpec((1, H), lambda i: (0, 0)),
              pl.BlockSpec((1, H), lambda i: (0, 0)),
              pl.BlockSpec((1, H), lambda i: (0, 0)),
              pl.BlockSpec((1, H), lambda i: (0, 0)),
              pl.BlockSpec((1, H), lambda i: (0, 0)),
              pl.BlockSpec((H, 3), lambda i: (0, 0)),
              pl.BlockSpec((1, 3), lambda i: (0, 0))],
    out_specs=(pl.BlockSpec((_NT, H), lambda i: (i, 0)),
               pl.BlockSpec((_NT, H), lambda i: (i, 0)),
               pl.BlockSpec((_NT, 3), lambda i: (i, 0))),
    out_shape=(jax.ShapeDtypeStruct((NN, H), jnp.float32),
               jax.ShapeDtypeStruct((NN, H), jnp.float32),
               jax.ShapeDtypeStruct((NN, 3), jnp.float32)),
)


def _ehead_body(a_ref, b_ref, b2_ref, o_ref):
    s = jnp.sum(a_ref[...] * b_ref[...], axis=1, keepdims=True)
    o_ref[...] = jax.nn.sigmoid(s + b2_ref[...])


_ehead = pl.pallas_call(
    _ehead_body,
    grid=(NE // _EH,),
    in_specs=[pl.BlockSpec((_EH, H), lambda i: (i, 0)),
              pl.BlockSpec((_EH, H), lambda i: (i, 0)),
              pl.BlockSpec((1, 1), lambda i: (0, 0))],
    out_specs=pl.BlockSpec((_EH, 1), lambda i: (i, 0)),
    out_shape=jax.ShapeDtypeStruct((NE, 1), jnp.float32),
)


def _updhead_body(a0_ref, a1_ref, out_ref, h0_ref, bc_ref, w1_ref, b1_ref,
                  mean_ref, var_ref, gam_ref, bet_ref, w2r_ref, w3_ref,
                  b3_ref, ybn_ref, yw_ref, ysig_ref):
    rst = a0_ref[...] + a1_ref[...] + out_ref[...] + bc_ref[...]
    temp = ALPHA * rst + (1.0 - ALPHA) * h0_ref[...]
    z = temp @ w1_ref[...] + b1_ref[...]
    out3 = jnp.maximum(BETA * z + (1.0 - BETA) * temp, 0.0)
    ybn = ((out3 - mean_ref[...]) * lax.rsqrt(var_ref[...] + 1e-5)
           * gam_ref[...] + bet_ref[...])
    ybn_ref[...] = ybn
    yw_ref[...] = ybn * w2r_ref[...]
    ysig_ref[...] = jax.nn.sigmoid(ybn @ w3_ref[...] + b3_ref[...])


_updhead = pl.pallas_call(
    _updhead_body,
    grid=(NN // _NT,),
    in_specs=[pl.BlockSpec((_NT, H), lambda i: (i, 0)),
              pl.BlockSpec((_NT, H), lambda i: (i, 0)),
              pl.BlockSpec((_NT, H), lambda i: (i, 0)),
              pl.BlockSpec((_NT, H), lambda i: (i, 0)),
              pl.BlockSpec((1, H), lambda i: (0, 0)),
              pl.BlockSpec((H, H), lambda i: (0, 0)),
              pl.BlockSpec((1, H), lambda i: (0, 0)),
              pl.BlockSpec((1, H), lambda i: (0, 0)),
              pl.BlockSpec((1, H), lambda i: (0, 0)),
              pl.BlockSpec((1, H), lambda i: (0, 0)),
              pl.BlockSpec((1, H), lambda i: (0, 0)),
              pl.BlockSpec((1, H), lambda i: (0, 0)),
              pl.BlockSpec((H, 3), lambda i: (0, 0)),
              pl.BlockSpec((1, 3), lambda i: (0, 0))],
    out_specs=(pl.BlockSpec((_NT, H), lambda i: (i, 0)),
               pl.BlockSpec((_NT, H), lambda i: (i, 0)),
               pl.BlockSpec((_NT, 3), lambda i: (i, 0))),
    out_shape=(jax.ShapeDtypeStruct((NN, H), jnp.float32),
               jax.ShapeDtypeStruct((NN, H), jnp.float32),
               jax.ShapeDtypeStruct((NN, 3), jnp.float32)),
)


def kernel(g, n_feat, e_feat, src_list, dst_list, W0, b0, We1, be1, We2, be2,
           b_conv, W1, b1, bn_gamma, bn_beta, bn_mean, bn_var, W3, b3, W2, b2):
    src = g[0].astype(jnp.int32)
    dst = g[1].astype(jnp.int32)
    sl = src_list.astype(jnp.int32)
    dl = dst_list.astype(jnp.int32)
    cols = jnp.arange(H * H, dtype=jnp.int32) // H
    r_mat = (cols[None, :] == jnp.arange(H, dtype=jnp.int32)[:, None]
             ).astype(jnp.float32)

    bvec = jnp.stack([b_conv, b1])
    out0 = _lin0(n_feat, W0, b0.reshape(1, H))
    out = out0
    xs = _sc_gather(out0, src)
    for step in range(NSTEPS):
        msg = _msg(xs, e_feat, We1, be1.reshape(1, EHID), We2,
                   be2.reshape(1, H * H), r_mat)
        if step < NSTEPS - 1:
            out, xs = _sc_sug(msg, dst, src, out, out0, W1, bvec)
        else:
            aggp = _sc_scatter(msg, dst)
    ybn, yw, ysig = _updhead(aggp[0], aggp[1], out, out0,
                             b_conv.reshape(1, H), W1, b1.reshape(1, H),
                             bn_mean.reshape(1, H), bn_var.reshape(1, H),
                             bn_gamma.reshape(1, H), bn_beta.reshape(1, H),
                             W2.reshape(1, H), W3, b3.reshape(1, 3))
    ga, gb = _sc_gather2(yw, ybn, sl, dl)
    ehop = _ehead(ga, gb, b2.reshape(1, 1))
    return (ysig, ehop)


# speedup vs baseline: 1.3129x; 1.3129x over previous
import jax
import jax.numpy as jnp
from jax import lax
from jax.experimental import pallas as pl
from jax.experimental.pallas import tpu as pltpu
from jax.experimental.pallas import tpu_sc as plsc

NN = 10000
NE = 160000
DIN = 128
H = 16
EHID = 64
NSTEPS = 3
ALPHA = 0.5
BETA = 1.0 / NSTEPS

NC = 2
NS = 16
NW = NC * NS
EPW = NE // NW
NPT = NN // NS
NEP = NE // 8

_SC_MESH = plsc.VectorSubcoreMesh(core_axis_name="c", subcore_axis_name="s")


def _sc_gather_body(table_hbm, idx_hbm, out_hbm, idx_v, rows_v, sem):
    wid = lax.axis_index("s") * NC + lax.axis_index("c")
    base = wid * EPW
    pltpu.sync_copy(idx_hbm.at[pl.ds(base, EPW)], idx_v)
    pltpu.async_copy(table_hbm.at[idx_v], rows_v, sem).wait()
    pltpu.sync_copy(rows_v, out_hbm.at[pl.ds(base, EPW)])


_sc_gather = pl.kernel(
    _sc_gather_body,
    out_type=jax.ShapeDtypeStruct((NE, H), jnp.float32),
    mesh=_SC_MESH,
    scratch_types=[
        pltpu.VMEM((EPW,), jnp.int32),
        pltpu.VMEM((EPW, H), jnp.float32),
        pltpu.SemaphoreType.DMA,
    ],
    compiler_params=pltpu.CompilerParams(use_tc_tiling_on_sc=False),
)


def _sc_gather2_body(ta_hbm, tb_hbm, ia_hbm, ib_hbm, oa_hbm, ob_hbm,
                     idx_v, rows_v, sem):
    wid = lax.axis_index("s") * NC + lax.axis_index("c")
    base = wid * EPW
    pltpu.sync_copy(ia_hbm.at[pl.ds(base, EPW)], idx_v)
    pltpu.async_copy(ta_hbm.at[idx_v], rows_v, sem).wait()
    pltpu.sync_copy(rows_v, oa_hbm.at[pl.ds(base, EPW)])
    pltpu.sync_copy(ib_hbm.at[pl.ds(base, EPW)], idx_v)
    pltpu.async_copy(tb_hbm.at[idx_v], rows_v, sem).wait()
    pltpu.sync_copy(rows_v, ob_hbm.at[pl.ds(base, EPW)])


_sc_gather2 = pl.kernel(
    _sc_gather2_body,
    out_type=(jax.ShapeDtypeStruct((NE, H), jnp.float32),
              jax.ShapeDtypeStruct((NE, H), jnp.float32)),
    mesh=_SC_MESH,
    scratch_types=[
        pltpu.VMEM((EPW,), jnp.int32),
        pltpu.VMEM((EPW, H), jnp.float32),
        pltpu.SemaphoreType.DMA,
    ],
    compiler_params=pltpu.CompilerParams(use_tc_tiling_on_sc=False),
)


def _sc_scatter_body(msg_hbm, dst_hbm, out_hbm, idx_v, msg_v, wb_v, acc_sh):
    cid = lax.axis_index("c")
    sid = lax.axis_index("s")
    wid = sid * NC + cid
    base = wid * EPW

    def zrow(i, carry):
        wb_v[i, :] = jnp.zeros((H,), jnp.float32)
        return carry

    lax.fori_loop(0, NPT, zrow, 0)
    pltpu.sync_copy(wb_v, acc_sh.at[pl.ds(sid * NPT, NPT)])
    plsc.subcore_barrier()
    pltpu.sync_copy(dst_hbm.at[pl.ds(base, EPW)], idx_v)
    pltpu.sync_copy(msg_hbm.at[pl.ds(base, EPW)], msg_v)
    pltpu.sync_copy(msg_v, acc_sh.at[idx_v], add=True)
    plsc.subcore_barrier()
    pltpu.sync_copy(acc_sh.at[pl.ds(sid * NPT, NPT)], wb_v)
    pltpu.sync_copy(wb_v, out_hbm.at[cid, pl.ds(sid * NPT, NPT)])


_sc_scatter = pl.kernel(
    _sc_scatter_body,
    out_type=jax.ShapeDtypeStruct((NC, NN, H), jnp.float32),
    mesh=_SC_MESH,
    scratch_types=[
        pltpu.VMEM((EPW,), jnp.int32),
        pltpu.VMEM((EPW, H), jnp.float32),
        pltpu.VMEM((NPT, H), jnp.float32),
        pltpu.VMEM_SHARED((NN, H), jnp.float32),
    ],
    compiler_params=pltpu.CompilerParams(use_tc_tiling_on_sc=False),
)


_CHK = 1000
_EPS = NE // NS
_NB_S = _EPS // _CHK
_NB_G = EPW // _CHK


def _sc_sug_body(msg_hbm, dst_hbm, src_hbm, old_hbm, h0_hbm, w1_hbm, bv_hbm,
                 outn_hbm, xs_hbm,
                 idx_v, buf_v, agg_v, old_v, h0_v, new_v, w1_v, bv_v,
                 acc_sh, tab_sh):
    cid = lax.axis_index("c")
    sid = lax.axis_index("s")
    wid = sid * NC + cid
    nbase = sid * NPT

    def zrow(i, c):
        new_v[i, :] = jnp.zeros((H,), jnp.float32)
        return c

    lax.fori_loop(0, NPT, zrow, 0)
    pltpu.sync_copy(new_v, acc_sh.at[pl.ds(nbase, NPT)])
    pltpu.sync_copy(w1_hbm, w1_v)
    pltpu.sync_copy(bv_hbm, bv_v)
    plsc.subcore_barrier()
    sbase = sid * _EPS
    for b in range(_NB_S):
        off = sbase + b * _CHK
        pltpu.sync_copy(dst_hbm.at[pl.ds(off, _CHK)], idx_v)
        pltpu.sync_copy(msg_hbm.at[pl.ds(off, _CHK)], buf_v)
        pltpu.sync_copy(buf_v, acc_sh.at[idx_v], add=True)
    plsc.subcore_barrier()
    pltpu.sync_copy(acc_sh.at[pl.ds(nbase, NPT)], agg_v)
    pltpu.sync_copy(old_hbm.at[pl.ds(nbase, NPT)], old_v)
    pltpu.sync_copy(h0_hbm.at[pl.ds(nbase, NPT)], h0_v)

    def urow(r, c):
        rst = agg_v[r, :] + old_v[r, :] + bv_v[0, :]
        temp = ALPHA * rst + (1.0 - ALPHA) * h0_v[r, :]
        z = BETA * bv_v[1, :] + (1.0 - BETA) * temp
        for i in range(H):
            z = z + (BETA * temp[i]) * w1_v[i, :]
        new_v[r, :] = jnp.maximum(z, 0.0)
        return c

    lax.fori_loop(0, NPT, urow, 0)
    pltpu.sync_copy(new_v, tab_sh.at[pl.ds(nbase, NPT)])

    @pl.when(cid == 0)
    def _():
        pltpu.sync_copy(new_v, outn_hbm.at[pl.ds(nbase, NPT)])

    plsc.subcore_barrier()
    gbase = wid * EPW
    for b in range(_NB_G):
        off = gbase + b * _CHK
        pltpu.sync_copy(src_hbm.at[pl.ds(off, _CHK)], idx_v)
        pltpu.sync_copy(tab_sh.at[idx_v], buf_v)
        pltpu.sync_copy(buf_v, xs_hbm.at[pl.ds(off, _CHK)])


_sc_sug = pl.kernel(
    _sc_sug_body,
    out_type=(jax.ShapeDtypeStruct((NN, H), jnp.float32),
              jax.ShapeDtypeStruct((NE, H), jnp.float32)),
    mesh=_SC_MESH,
    scratch_types=[
        pltpu.VMEM((_CHK,), jnp.int32),
        pltpu.VMEM((_CHK, H), jnp.float32),
        pltpu.VMEM((NPT, H), jnp.float32),
        pltpu.VMEM((NPT, H), jnp.float32),
        pltpu.VMEM((NPT, H), jnp.float32),
        pltpu.VMEM((NPT, H), jnp.float32),
        pltpu.VMEM((H, H), jnp.float32),
        pltpu.VMEM((2, H), jnp.float32),
        pltpu.VMEM_SHARED((NN, H), jnp.float32),
        pltpu.VMEM_SHARED((NN, H), jnp.float32),
    ],
    compiler_params=pltpu.CompilerParams(use_tc_tiling_on_sc=False),
)


_NT = 2000
_ET = 8000
_EH = 16000


def _lin0_body(x_ref, w_ref, b_ref, o_ref):
    o_ref[...] = jnp.maximum(x_ref[...] @ w_ref[...] + b_ref[...], 0.0)


_lin0 = pl.pallas_call(
    _lin0_body,
    grid=(NN // _NT,),
    in_specs=[pl.BlockSpec((_NT, DIN), lambda i: (i, 0)),
              pl.BlockSpec((DIN, H), lambda i: (0, 0)),
              pl.BlockSpec((1, H), lambda i: (0, 0))],
    out_specs=pl.BlockSpec((_NT, H), lambda i: (i, 0)),
    out_shape=jax.ShapeDtypeStruct((NN, H), jnp.float32),
)


def _msg_body(xs_ref, ef_ref, we1_ref, be1_ref, we2_ref, be2_ref, r_ref,
              o_ref):
    xs_p = xs_ref[...]
    ef_p = ef_ref[...]
    for j in range(8):
        xs_j = xs_p[:, 16 * j:16 * j + 16]
        ef_j = ef_p[:, 16 * j:16 * j + 16]
        u = jnp.maximum(ef_j @ we1_ref[...] + be1_ref[...], 0.0)
        ew = u @ we2_ref[...] + be2_ref[...]
        p = (xs_j @ r_ref[...]) * ew
        p = p[:, :128] + p[:, 128:]
        p = p[:, :64] + p[:, 64:]
        p = p[:, :32] + p[:, 32:]
        o_ref[:, 16 * j:16 * j + 16] = p[:, :16] + p[:, 16:]


_msg = pl.pallas_call(
    _msg_body,
    grid=(NE // _ET,),
    in_specs=[pl.BlockSpec((_ET // 8, 128), lambda i: (i, 0)),
              pl.BlockSpec((_ET // 8, 128), lambda i: (i, 0)),
              pl.BlockSpec((H, EHID), lambda i: (0, 0)),
              pl.BlockSpec((1, EHID), lambda i: (0, 0)),
              pl.BlockSpec((EHID, H * H), lambda i: (0, 0)),
              pl.BlockSpec((1, H * H), lambda i: (0, 0)),
              pl.BlockSpec((H, H * H), lambda i: (0, 0))],
    out_specs=pl.BlockSpec((_ET // 8, 128), lambda i: (i, 0)),
    out_shape=jax.ShapeDtypeStruct((NEP, 128), jnp.float32),
)


def _upd_body(a0_ref, a1_ref, out_ref, h0_ref, bc_ref, w1_ref, b1_ref, o_ref):
    rst = a0_ref[...] + a1_ref[...] + out_ref[...] + bc_ref[...]
    temp = ALPHA * rst + (1.0 - ALPHA) * h0_ref[...]
    z = temp @ w1_ref[...] + b1_ref[...]
    o_ref[...] = jnp.maximum(BETA * z + (1.0 - BETA) * temp, 0.0)


_upd = pl.pallas_call(
    _upd_body,
    grid=(NN // _NT,),
    in_specs=[pl.BlockSpec((_NT, H), lambda i: (i, 0)),
              pl.BlockSpec((_NT, H), lambda i: (i, 0)),
              pl.BlockSpec((_NT, H), lambda i: (i, 0)),
              pl.BlockSpec((_NT, H), lambda i: (i, 0)),
              pl.BlockSpec((1, H), lambda i: (0, 0)),
              pl.BlockSpec((H, H), lambda i: (0, 0)),
              pl.BlockSpec((1, H), lambda i: (0, 0))],
    out_specs=pl.BlockSpec((_NT, H), lambda i: (i, 0)),
    out_shape=jax.ShapeDtypeStruct((NN, H), jnp.float32),
)


def _head_body(out_ref, mean_ref, var_ref, gam_ref, bet_ref, w2r_ref, w3_ref,
               b3_ref, ybn_ref, yw_ref, ysig_ref):
    ybn = ((out_ref[...] - mean_ref[...]) * lax.rsqrt(var_ref[...] + 1e-5)
           * gam_ref[...] + bet_ref[...])
    ybn_ref[...] = ybn
    yw_ref[...] = ybn * w2r_ref[...]
    ysig_ref[...] = jax.nn.sigmoid(ybn @ w3_ref[...] + b3_ref[...])


_head = pl.pallas_call(
    _head_body,
    grid=(NN // _NT,),
    in_specs=[pl.BlockSpec((_NT, H), lambda i: (i, 0)),
              pl.BlockSpec((1, H), lambda i: (0, 0)),
              pl.BlockSpec((1, H), lambda i: (0, 0)),
              pl.BlockSpec((1, H), lambda i: (0, 0)),
              pl.BlockSpec((1, H), lambda i: (0, 0)),
              pl.BlockSpec((1, H), lambda i: (0, 0)),
              pl.BlockSpec((H, 3), lambda i: (0, 0)),
              pl.BlockSpec((1, 3), lambda i: (0, 0))],
    out_specs=(pl.BlockSpec((_NT, H), lambda i: (i, 0)),
               pl.BlockSpec((_NT, H), lambda i: (i, 0)),
               pl.BlockSpec((_NT, 3), lambda i: (i, 0))),
    out_shape=(jax.ShapeDtypeStruct((NN, H), jnp.float32),
               jax.ShapeDtypeStruct((NN, H), jnp.float32),
               jax.ShapeDtypeStruct((NN, 3), jnp.float32)),
)


def _ehead_body(a_ref, b_ref, g_ref, b2_ref, o_ref):
    s = (a_ref[...] * b_ref[...]) @ g_ref[...]
    o_ref[...] = jax.nn.sigmoid(s + b2_ref[...])


_ehead = pl.pallas_call(
    _ehead_body,
    grid=(NE // _EH,),
    in_specs=[pl.BlockSpec((_EH // 8, 128), lambda i: (i, 0)),
              pl.BlockSpec((_EH // 8, 128), lambda i: (i, 0)),
              pl.BlockSpec((128, 8), lambda i: (0, 0)),
              pl.BlockSpec((1, 1), lambda i: (0, 0))],
    out_specs=pl.BlockSpec((_EH // 8, 8), lambda i: (i, 0)),
    out_shape=jax.ShapeDtypeStruct((NEP, 8), jnp.float32),
)


def _updhead_body(a0_ref, a1_ref, out_ref, h0_ref, bc_ref, w1_ref, b1_ref,
                  mean_ref, var_ref, gam_ref, bet_ref, w2r_ref, w3_ref,
                  b3_ref, ybn_ref, yw_ref, ysig_ref):
    rst = a0_ref[...] + a1_ref[...] + out_ref[...] + bc_ref[...]
    temp = ALPHA * rst + (1.0 - ALPHA) * h0_ref[...]
    z = temp @ w1_ref[...] + b1_ref[...]
    out3 = jnp.maximum(BETA * z + (1.0 - BETA) * temp, 0.0)
    ybn = ((out3 - mean_ref[...]) * lax.rsqrt(var_ref[...] + 1e-5)
           * gam_ref[...] + bet_ref[...])
    ybn_ref[...] = ybn
    yw_ref[...] = ybn * w2r_ref[...]
    ysig_ref[...] = jax.nn.sigmoid(ybn @ w3_ref[...] + b3_ref[...])


_updhead = pl.pallas_call(
    _updhead_body,
    grid=(NN // _NT,),
    in_specs=[pl.BlockSpec((_NT, H), lambda i: (i, 0)),
              pl.BlockSpec((_NT, H), lambda i: (i, 0)),
              pl.BlockSpec((_NT, H), lambda i: (i, 0)),
              pl.BlockSpec((_NT, H), lambda i: (i, 0)),
              pl.BlockSpec((1, H), lambda i: (0, 0)),
              pl.BlockSpec((H, H), lambda i: (0, 0)),
              pl.BlockSpec((1, H), lambda i: (0, 0)),
              pl.BlockSpec((1, H), lambda i: (0, 0)),
              pl.BlockSpec((1, H), lambda i: (0, 0)),
              pl.BlockSpec((1, H), lambda i: (0, 0)),
              pl.BlockSpec((1, H), lambda i: (0, 0)),
              pl.BlockSpec((1, H), lambda i: (0, 0)),
              pl.BlockSpec((H, 3), lambda i: (0, 0)),
              pl.BlockSpec((1, 3), lambda i: (0, 0))],
    out_specs=(pl.BlockSpec((_NT, H), lambda i: (i, 0)),
               pl.BlockSpec((_NT, H), lambda i: (i, 0)),
               pl.BlockSpec((_NT, 3), lambda i: (i, 0))),
    out_shape=(jax.ShapeDtypeStruct((NN, H), jnp.float32),
               jax.ShapeDtypeStruct((NN, H), jnp.float32),
               jax.ShapeDtypeStruct((NN, 3), jnp.float32)),
)


def kernel(g, n_feat, e_feat, src_list, dst_list, W0, b0, We1, be1, We2, be2,
           b_conv, W1, b1, bn_gamma, bn_beta, bn_mean, bn_var, W3, b3, W2, b2):
    src = g[0].astype(jnp.int32)
    dst = g[1].astype(jnp.int32)
    sl = src_list.astype(jnp.int32)
    dl = dst_list.astype(jnp.int32)
    cols = jnp.arange(H * H, dtype=jnp.int32) // H
    r_mat = (cols[None, :] == jnp.arange(H, dtype=jnp.int32)[:, None]
             ).astype(jnp.float32)
    lanes = jnp.arange(128, dtype=jnp.int32) // H
    g_mat = (lanes[:, None] == jnp.arange(8, dtype=jnp.int32)[None, :]
             ).astype(jnp.float32)
    ef_p = e_feat.reshape(NEP, 128)

    bvec = jnp.stack([b_conv, b1])
    out0 = _lin0(n_feat, W0, b0.reshape(1, H))
    out = out0
    xs = _sc_gather(out0, src)
    for step in range(NSTEPS):
        msg_p = _msg(xs.reshape(NEP, 128), ef_p, We1, be1.reshape(1, EHID),
                     We2, be2.reshape(1, H * H), r_mat)
        msg = msg_p.reshape(NE, H)
        if step < NSTEPS - 1:
            out, xs = _sc_sug(msg, dst, src, out, out0, W1, bvec)
        else:
            aggp = _sc_scatter(msg, dst)
    ybn, yw, ysig = _updhead(aggp[0], aggp[1], out, out0,
                             b_conv.reshape(1, H), W1, b1.reshape(1, H),
                             bn_mean.reshape(1, H), bn_var.reshape(1, H),
                             bn_gamma.reshape(1, H), bn_beta.reshape(1, H),
                             W2.reshape(1, H), W3, b3.reshape(1, 3))
    ga, gb = _sc_gather2(yw, ybn, sl, dl)
    ehop_p = _ehead(ga.reshape(NEP, 128), gb.reshape(NEP, 128), g_mat,
                    b2.reshape(1, 1))
    return (ysig, ehop_p.reshape(NE, 1))


# speedup vs baseline: 1.8725x; 1.4263x over previous
import jax
import jax.numpy as jnp
from jax import lax
from jax.experimental import pallas as pl
from jax.experimental.pallas import tpu as pltpu
from jax.experimental.pallas import tpu_sc as plsc

NN = 10000
NE = 160000
DIN = 128
H = 16
EHID = 64
NSTEPS = 3
ALPHA = 0.5
BETA = 1.0 / NSTEPS

NC = 2
NS = 16
NW = NC * NS
EPW = NE // NW
NPT = NN // NS
NEP = NE // 8

_SC_MESH = plsc.VectorSubcoreMesh(core_axis_name="c", subcore_axis_name="s")


def _sc_gather_body(table_hbm, idx_hbm, out_hbm, idx_v, rows_v, sem):
    wid = lax.axis_index("s") * NC + lax.axis_index("c")
    base = wid * EPW
    pltpu.sync_copy(idx_hbm.at[pl.ds(base, EPW)], idx_v)
    pltpu.async_copy(table_hbm.at[idx_v], rows_v, sem).wait()
    pltpu.sync_copy(rows_v, out_hbm.at[pl.ds(base, EPW)])


_sc_gather = pl.kernel(
    _sc_gather_body,
    out_type=jax.ShapeDtypeStruct((NE, H), jnp.float32),
    mesh=_SC_MESH,
    scratch_types=[
        pltpu.VMEM((EPW,), jnp.int32),
        pltpu.VMEM((EPW, H), jnp.float32),
        pltpu.SemaphoreType.DMA,
    ],
    compiler_params=pltpu.CompilerParams(use_tc_tiling_on_sc=False),
)


def _sc_gather2_body(ta_hbm, tb_hbm, ia_hbm, ib_hbm, oa_hbm, ob_hbm,
                     idx_v, rows_v, sem):
    wid = lax.axis_index("s") * NC + lax.axis_index("c")
    base = wid * EPW
    pltpu.sync_copy(ia_hbm.at[pl.ds(base, EPW)], idx_v)
    pltpu.async_copy(ta_hbm.at[idx_v], rows_v, sem).wait()
    pltpu.sync_copy(rows_v, oa_hbm.at[pl.ds(base, EPW)])
    pltpu.sync_copy(ib_hbm.at[pl.ds(base, EPW)], idx_v)
    pltpu.async_copy(tb_hbm.at[idx_v], rows_v, sem).wait()
    pltpu.sync_copy(rows_v, ob_hbm.at[pl.ds(base, EPW)])


_sc_gather2 = pl.kernel(
    _sc_gather2_body,
    out_type=(jax.ShapeDtypeStruct((NE, H), jnp.float32),
              jax.ShapeDtypeStruct((NE, H), jnp.float32)),
    mesh=_SC_MESH,
    scratch_types=[
        pltpu.VMEM((EPW,), jnp.int32),
        pltpu.VMEM((EPW, H), jnp.float32),
        pltpu.SemaphoreType.DMA,
    ],
    compiler_params=pltpu.CompilerParams(use_tc_tiling_on_sc=False),
)


def _sc_scatter_body(msg_hbm, dst_hbm, out_hbm, idx_v, msg_v, wb_v, acc_sh):
    cid = lax.axis_index("c")
    sid = lax.axis_index("s")
    wid = sid * NC + cid
    base = wid * EPW

    def zrow(i, carry):
        wb_v[i, :] = jnp.zeros((H,), jnp.float32)
        return carry

    lax.fori_loop(0, NPT, zrow, 0)
    pltpu.sync_copy(wb_v, acc_sh.at[pl.ds(sid * NPT, NPT)])
    plsc.subcore_barrier()
    pltpu.sync_copy(dst_hbm.at[pl.ds(base, EPW)], idx_v)
    pltpu.sync_copy(msg_hbm.at[pl.ds(base, EPW)], msg_v)
    pltpu.sync_copy(msg_v, acc_sh.at[idx_v], add=True)
    plsc.subcore_barrier()
    pltpu.sync_copy(acc_sh.at[pl.ds(sid * NPT, NPT)], wb_v)
    pltpu.sync_copy(wb_v, out_hbm.at[cid, pl.ds(sid * NPT, NPT)])


_sc_scatter = pl.kernel(
    _sc_scatter_body,
    out_type=jax.ShapeDtypeStruct((NC, NN, H), jnp.float32),
    mesh=_SC_MESH,
    scratch_types=[
        pltpu.VMEM((EPW,), jnp.int32),
        pltpu.VMEM((EPW, H), jnp.float32),
        pltpu.VMEM((NPT, H), jnp.float32),
        pltpu.VMEM_SHARED((NN, H), jnp.float32),
    ],
    compiler_params=pltpu.CompilerParams(use_tc_tiling_on_sc=False),
)


_CHK = 1000
_EPS = NE // NS
_NB_S = _EPS // _CHK
_NB_G = EPW // _CHK


def _sc_sug_body(msg_hbm, dst_hbm, src_hbm, old_hbm, h0_hbm, w1_hbm, bv_hbm,
                 outn_hbm, xs_hbm,
                 idx_v, buf_v, agg_v, old_v, h0_v, new_v, w1_v, bv_v,
                 acc_sh, tab_sh):
    cid = lax.axis_index("c")
    sid = lax.axis_index("s")
    wid = sid * NC + cid
    nbase = sid * NPT

    def zrow(i, c):
        new_v[i, :] = jnp.zeros((H,), jnp.float32)
        return c

    lax.fori_loop(0, NPT, zrow, 0)
    pltpu.sync_copy(new_v, acc_sh.at[pl.ds(nbase, NPT)])
    pltpu.sync_copy(w1_hbm, w1_v)
    pltpu.sync_copy(bv_hbm, bv_v)
    plsc.subcore_barrier()
    sbase = sid * _EPS
    for b in range(_NB_S):
        off = sbase + b * _CHK
        pltpu.sync_copy(dst_hbm.at[pl.ds(off, _CHK)], idx_v)
        pltpu.sync_copy(msg_hbm.at[pl.ds(off, _CHK)], buf_v)
        pltpu.sync_copy(buf_v, acc_sh.at[idx_v], add=True)
    plsc.subcore_barrier()
    pltpu.sync_copy(acc_sh.at[pl.ds(nbase, NPT)], agg_v)
    pltpu.sync_copy(old_hbm.at[pl.ds(nbase, NPT)], old_v)
    pltpu.sync_copy(h0_hbm.at[pl.ds(nbase, NPT)], h0_v)

    def urow(r, c):
        rst = agg_v[r, :] + old_v[r, :] + bv_v[0, :]
        temp = ALPHA * rst + (1.0 - ALPHA) * h0_v[r, :]
        z = BETA * bv_v[1, :] + (1.0 - BETA) * temp
        for i in range(H):
            z = z + (BETA * temp[i]) * w1_v[i, :]
        new_v[r, :] = jnp.maximum(z, 0.0)
        return c

    lax.fori_loop(0, NPT, urow, 0)
    pltpu.sync_copy(new_v, tab_sh.at[pl.ds(nbase, NPT)])

    @pl.when(cid == 0)
    def _():
        pltpu.sync_copy(new_v, outn_hbm.at[pl.ds(nbase, NPT)])

    plsc.subcore_barrier()
    gbase = wid * EPW
    for b in range(_NB_G):
        off = gbase + b * _CHK
        pltpu.sync_copy(src_hbm.at[pl.ds(off, _CHK)], idx_v)
        pltpu.sync_copy(tab_sh.at[idx_v], buf_v)
        pltpu.sync_copy(buf_v, xs_hbm.at[pl.ds(off, _CHK)])


_sc_sug = pl.kernel(
    _sc_sug_body,
    out_type=(jax.ShapeDtypeStruct((NN, H), jnp.float32),
              jax.ShapeDtypeStruct((NE, H), jnp.float32)),
    mesh=_SC_MESH,
    scratch_types=[
        pltpu.VMEM((_CHK,), jnp.int32),
        pltpu.VMEM((_CHK, H), jnp.float32),
        pltpu.VMEM((NPT, H), jnp.float32),
        pltpu.VMEM((NPT, H), jnp.float32),
        pltpu.VMEM((NPT, H), jnp.float32),
        pltpu.VMEM((NPT, H), jnp.float32),
        pltpu.VMEM((H, H), jnp.float32),
        pltpu.VMEM((2, H), jnp.float32),
        pltpu.VMEM_SHARED((NN, H), jnp.float32),
        pltpu.VMEM_SHARED((NN, H), jnp.float32),
    ],
    compiler_params=pltpu.CompilerParams(use_tc_tiling_on_sc=False),
)


_NT = 2000
_ET = 8000
_EH = 16000


def _lin0_body(x_ref, w_ref, b_ref, o_ref):
    o_ref[...] = jnp.maximum(x_ref[...] @ w_ref[...] + b_ref[...], 0.0)


_lin0 = pl.pallas_call(
    _lin0_body,
    grid=(NN // _NT,),
    in_specs=[pl.BlockSpec((_NT, DIN), lambda i: (i, 0)),
              pl.BlockSpec((DIN, H), lambda i: (0, 0)),
              pl.BlockSpec((1, H), lambda i: (0, 0))],
    out_specs=pl.BlockSpec((_NT, H), lambda i: (i, 0)),
    out_shape=jax.ShapeDtypeStruct((NN, H), jnp.float32),
)


def _msg_body(xs_ref, ef_ref, we1_ref, be1_ref, we2_ref, be2_ref, r_ref,
              o_ref):
    xs_t = xs_ref[...].T
    ef_t = ef_ref[...].T
    we1_t = we1_ref[...].T
    we2_t = we2_ref[...].T
    r_t = r_ref[...].T
    be1_t = be1_ref[...].reshape(EHID, 1)
    be2_t = be2_ref[...].reshape(H * H, 1)
    mt = []
    for j in range(8):
        ef_j = ef_t[16 * j:16 * j + 16, :]
        xs_j = xs_t[16 * j:16 * j + 16, :]
        u = jnp.maximum(we1_t @ ef_j + be1_t, 0.0)
        ew = we2_t @ u + be2_t
        p = (r_t @ xs_j) * ew
        p = p[:128, :] + p[128:, :]
        p = p[:64, :] + p[64:, :]
        p = p[:32, :] + p[32:, :]
        mt.append(p[:16, :] + p[16:, :])
    o_ref[...] = jnp.concatenate(mt, axis=0).T


_msg = pl.pallas_call(
    _msg_body,
    grid=(NE // _ET,),
    in_specs=[pl.BlockSpec((_ET // 8, 128), lambda i: (i, 0)),
              pl.BlockSpec((_ET // 8, 128), lambda i: (i, 0)),
              pl.BlockSpec((H, EHID), lambda i: (0, 0)),
              pl.BlockSpec((1, EHID), lambda i: (0, 0)),
              pl.BlockSpec((EHID, H * H), lambda i: (0, 0)),
              pl.BlockSpec((1, H * H), lambda i: (0, 0)),
              pl.BlockSpec((H, H * H), lambda i: (0, 0))],
    out_specs=pl.BlockSpec((_ET // 8, 128), lambda i: (i, 0)),
    out_shape=jax.ShapeDtypeStruct((NEP, 128), jnp.float32),
)


def _upd_body(a0_ref, a1_ref, out_ref, h0_ref, bc_ref, w1_ref, b1_ref, o_ref):
    rst = a0_ref[...] + a1_ref[...] + out_ref[...] + bc_ref[...]
    temp = ALPHA * rst + (1.0 - ALPHA) * h0_ref[...]
    z = temp @ w1_ref[...] + b1_ref[...]
    o_ref[...] = jnp.maximum(BETA * z + (1.0 - BETA) * temp, 0.0)


_upd = pl.pallas_call(
    _upd_body,
    grid=(NN // _NT,),
    in_specs=[pl.BlockSpec((_NT, H), lambda i: (i, 0)),
              pl.BlockSpec((_NT, H), lambda i: (i, 0)),
              pl.BlockSpec((_NT, H), lambda i: (i, 0)),
              pl.BlockSpec((_NT, H), lambda i: (i, 0)),
              pl.BlockSpec((1, H), lambda i: (0, 0)),
              pl.BlockSpec((H, H), lambda i: (0, 0)),
              pl.BlockSpec((1, H), lambda i: (0, 0))],
    out_specs=pl.BlockSpec((_NT, H), lambda i: (i, 0)),
    out_shape=jax.ShapeDtypeStruct((NN, H), jnp.float32),
)


def _head_body(out_ref, mean_ref, var_ref, gam_ref, bet_ref, w2r_ref, w3_ref,
               b3_ref, ybn_ref, yw_ref, ysig_ref):
    ybn = ((out_ref[...] - mean_ref[...]) * lax.rsqrt(var_ref[...] + 1e-5)
           * gam_ref[...] + bet_ref[...])
    ybn_ref[...] = ybn
    yw_ref[...] = ybn * w2r_ref[...]
    ysig_ref[...] = jax.nn.sigmoid(ybn @ w3_ref[...] + b3_ref[...])


_head = pl.pallas_call(
    _head_body,
    grid=(NN // _NT,),
    in_specs=[pl.BlockSpec((_NT, H), lambda i: (i, 0)),
              pl.BlockSpec((1, H), lambda i: (0, 0)),
              pl.BlockSpec((1, H), lambda i: (0, 0)),
              pl.BlockSpec((1, H), lambda i: (0, 0)),
              pl.BlockSpec((1, H), lambda i: (0, 0)),
              pl.BlockSpec((1, H), lambda i: (0, 0)),
              pl.BlockSpec((H, 3), lambda i: (0, 0)),
              pl.BlockSpec((1, 3), lambda i: (0, 0))],
    out_specs=(pl.BlockSpec((_NT, H), lambda i: (i, 0)),
               pl.BlockSpec((_NT, H), lambda i: (i, 0)),
               pl.BlockSpec((_NT, 3), lambda i: (i, 0))),
    out_shape=(jax.ShapeDtypeStruct((NN, H), jnp.float32),
               jax.ShapeDtypeStruct((NN, H), jnp.float32),
               jax.ShapeDtypeStruct((NN, 3), jnp.float32)),
)


def _ehead_body(a_ref, b_ref, g_ref, b2_ref, o_ref):
    s = (a_ref[...] * b_ref[...]) @ g_ref[...]
    o_ref[...] = jax.nn.sigmoid(s + b2_ref[...])


_ehead = pl.pallas_call(
    _ehead_body,
    grid=(NE // _EH,),
    in_specs=[pl.BlockSpec((_EH // 8, 128), lambda i: (i, 0)),
              pl.BlockSpec((_EH // 8, 128), lambda i: (i, 0)),
              pl.BlockSpec((128, 8), lambda i: (0, 0)),
              pl.BlockSpec((1, 1), lambda i: (0, 0))],
    out_specs=pl.BlockSpec((_EH // 8, 8), lambda i: (i, 0)),
    out_shape=jax.ShapeDtypeStruct((NEP, 8), jnp.float32),
)


def _updhead_body(a0_ref, a1_ref, out_ref, h0_ref, bc_ref, w1_ref, b1_ref,
                  mean_ref, var_ref, gam_ref, bet_ref, w2r_ref, w3_ref,
                  b3_ref, ybn_ref, yw_ref, ysig_ref):
    rst = a0_ref[...] + a1_ref[...] + out_ref[...] + bc_ref[...]
    temp = ALPHA * rst + (1.0 - ALPHA) * h0_ref[...]
    z = temp @ w1_ref[...] + b1_ref[...]
    out3 = jnp.maximum(BETA * z + (1.0 - BETA) * temp, 0.0)
    ybn = ((out3 - mean_ref[...]) * lax.rsqrt(var_ref[...] + 1e-5)
           * gam_ref[...] + bet_ref[...])
    ybn_ref[...] = ybn
    yw_ref[...] = ybn * w2r_ref[...]
    ysig_ref[...] = jax.nn.sigmoid(ybn @ w3_ref[...] + b3_ref[...])


_updhead = pl.pallas_call(
    _updhead_body,
    grid=(NN // _NT,),
    in_specs=[pl.BlockSpec((_NT, H), lambda i: (i, 0)),
              pl.BlockSpec((_NT, H), lambda i: (i, 0)),
              pl.BlockSpec((_NT, H), lambda i: (i, 0)),
              pl.BlockSpec((_NT, H), lambda i: (i, 0)),
              pl.BlockSpec((1, H), lambda i: (0, 0)),
              pl.BlockSpec((H, H), lambda i: (0, 0)),
              pl.BlockSpec((1, H), lambda i: (0, 0)),
              pl.BlockSpec((1, H), lambda i: (0, 0)),
              pl.BlockSpec((1, H), lambda i: (0, 0)),
              pl.BlockSpec((1, H), lambda i: (0, 0)),
              pl.BlockSpec((1, H), lambda i: (0, 0)),
              pl.BlockSpec((1, H), lambda i: (0, 0)),
              pl.BlockSpec((H, 3), lambda i: (0, 0)),
              pl.BlockSpec((1, 3), lambda i: (0, 0))],
    out_specs=(pl.BlockSpec((_NT, H), lambda i: (i, 0)),
               pl.BlockSpec((_NT, H), lambda i: (i, 0)),
               pl.BlockSpec((_NT, 3), lambda i: (i, 0))),
    out_shape=(jax.ShapeDtypeStruct((NN, H), jnp.float32),
               jax.ShapeDtypeStruct((NN, H), jnp.float32),
               jax.ShapeDtypeStruct((NN, 3), jnp.float32)),
)


def kernel(g, n_feat, e_feat, src_list, dst_list, W0, b0, We1, be1, We2, be2,
           b_conv, W1, b1, bn_gamma, bn_beta, bn_mean, bn_var, W3, b3, W2, b2):
    src = g[0].astype(jnp.int32)
    dst = g[1].astype(jnp.int32)
    sl = src_list.astype(jnp.int32)
    dl = dst_list.astype(jnp.int32)
    cols = jnp.arange(H * H, dtype=jnp.int32) // H
    r_mat = (cols[None, :] == jnp.arange(H, dtype=jnp.int32)[:, None]
             ).astype(jnp.float32)
    lanes = jnp.arange(128, dtype=jnp.int32) // H
    g_mat = (lanes[:, None] == jnp.arange(8, dtype=jnp.int32)[None, :]
             ).astype(jnp.float32)
    ef_p = e_feat.reshape(NEP, 128)

    bvec = jnp.stack([b_conv, b1])
    out0 = _lin0(n_feat, W0, b0.reshape(1, H))
    out = out0
    xs = _sc_gather(out0, src)
    for step in range(NSTEPS):
        msg_p = _msg(xs.reshape(NEP, 128), ef_p, We1, be1.reshape(1, EHID),
                     We2, be2.reshape(1, H * H), r_mat)
        msg = msg_p.reshape(NE, H)
        if step < NSTEPS - 1:
            out, xs = _sc_sug(msg, dst, src, out, out0, W1, bvec)
        else:
            aggp = _sc_scatter(msg, dst)
    ybn, yw, ysig = _updhead(aggp[0], aggp[1], out, out0,
                             b_conv.reshape(1, H), W1, b1.reshape(1, H),
                             bn_mean.reshape(1, H), bn_var.reshape(1, H),
                             bn_gamma.reshape(1, H), bn_beta.reshape(1, H),
                             W2.reshape(1, H), W3, b3.reshape(1, 3))
    ga, gb = _sc_gather2(yw, ybn, sl, dl)
    ehop_p = _ehead(ga.reshape(NEP, 128), gb.reshape(NEP, 128), g_mat,
                    b2.reshape(1, 1))
    return (ysig, ehop_p.reshape(NE, 1))


# speedup vs baseline: 1.9886x; 1.0620x over previous
import jax
import jax.numpy as jnp
from jax import lax
from jax.experimental import pallas as pl
from jax.experimental.pallas import tpu as pltpu
from jax.experimental.pallas import tpu_sc as plsc

NN = 10000
NE = 160000
DIN = 128
H = 16
EHID = 64
NSTEPS = 3
ALPHA = 0.5
BETA = 1.0 / NSTEPS

NC = 2
NS = 16
NW = NC * NS
EPW = NE // NW
NPT = NN // NS
NEP = NE // 8

_SC_MESH = plsc.VectorSubcoreMesh(core_axis_name="c", subcore_axis_name="s")


def _sc_gather_body(table_hbm, idx_hbm, out_hbm, idx_v, rows_v, sem):
    wid = lax.axis_index("s") * NC + lax.axis_index("c")
    base = wid * EPW
    pltpu.sync_copy(idx_hbm.at[pl.ds(base, EPW)], idx_v)
    pltpu.async_copy(table_hbm.at[idx_v], rows_v, sem).wait()
    pltpu.sync_copy(rows_v, out_hbm.at[pl.ds(base, EPW)])


_sc_gather = pl.kernel(
    _sc_gather_body,
    out_type=jax.ShapeDtypeStruct((NE, H), jnp.float32),
    mesh=_SC_MESH,
    scratch_types=[
        pltpu.VMEM((EPW,), jnp.int32),
        pltpu.VMEM((EPW, H), jnp.float32),
        pltpu.SemaphoreType.DMA,
    ],
    compiler_params=pltpu.CompilerParams(use_tc_tiling_on_sc=False),
)


def _sc_gather2_body(ta_hbm, tb_hbm, ia_hbm, ib_hbm, oa_hbm, ob_hbm,
                     idx_v, rows_v, sem):
    wid = lax.axis_index("s") * NC + lax.axis_index("c")
    base = wid * EPW
    pltpu.sync_copy(ia_hbm.at[pl.ds(base, EPW)], idx_v)
    pltpu.async_copy(ta_hbm.at[idx_v], rows_v, sem).wait()
    pltpu.sync_copy(rows_v, oa_hbm.at[pl.ds(base, EPW)])
    pltpu.sync_copy(ib_hbm.at[pl.ds(base, EPW)], idx_v)
    pltpu.async_copy(tb_hbm.at[idx_v], rows_v, sem).wait()
    pltpu.sync_copy(rows_v, ob_hbm.at[pl.ds(base, EPW)])


_sc_gather2 = pl.kernel(
    _sc_gather2_body,
    out_type=(jax.ShapeDtypeStruct((NE, H), jnp.float32),
              jax.ShapeDtypeStruct((NE, H), jnp.float32)),
    mesh=_SC_MESH,
    scratch_types=[
        pltpu.VMEM((EPW,), jnp.int32),
        pltpu.VMEM((EPW, H), jnp.float32),
        pltpu.SemaphoreType.DMA,
    ],
    compiler_params=pltpu.CompilerParams(use_tc_tiling_on_sc=False),
)


def _sc_scatter_body(msg_hbm, dst_hbm, out_hbm, idx_v, msg_v, wb_v, acc_sh):
    cid = lax.axis_index("c")
    sid = lax.axis_index("s")
    wid = sid * NC + cid
    base = wid * EPW

    def zrow(i, carry):
        wb_v[i, :] = jnp.zeros((H,), jnp.float32)
        return carry

    lax.fori_loop(0, NPT, zrow, 0)
    pltpu.sync_copy(wb_v, acc_sh.at[pl.ds(sid * NPT, NPT)])
    plsc.subcore_barrier()
    pltpu.sync_copy(dst_hbm.at[pl.ds(base, EPW)], idx_v)
    pltpu.sync_copy(msg_hbm.at[pl.ds(base, EPW)], msg_v)
    pltpu.sync_copy(msg_v, acc_sh.at[idx_v], add=True)
    plsc.subcore_barrier()
    pltpu.sync_copy(acc_sh.at[pl.ds(sid * NPT, NPT)], wb_v)
    pltpu.sync_copy(wb_v, out_hbm.at[cid, pl.ds(sid * NPT, NPT)])


_sc_scatter = pl.kernel(
    _sc_scatter_body,
    out_type=jax.ShapeDtypeStruct((NC, NN, H), jnp.float32),
    mesh=_SC_MESH,
    scratch_types=[
        pltpu.VMEM((EPW,), jnp.int32),
        pltpu.VMEM((EPW, H), jnp.float32),
        pltpu.VMEM((NPT, H), jnp.float32),
        pltpu.VMEM_SHARED((NN, H), jnp.float32),
    ],
    compiler_params=pltpu.CompilerParams(use_tc_tiling_on_sc=False),
)


_CHK = 1000
_EPS = NE // NS
_NB_S = _EPS // _CHK
_NB_G = EPW // _CHK


def _sc_sug_body(msg_hbm, dst_hbm, src_hbm, old_hbm, h0_hbm, w1_hbm, bv_hbm,
                 outn_hbm, xs_hbm,
                 idx_v, buf_v, agg_v, old_v, h0_v, new_v, w1_v, bv_v,
                 acc_sh, tab_sh):
    cid = lax.axis_index("c")
    sid = lax.axis_index("s")
    wid = sid * NC + cid
    nbase = sid * NPT

    def zrow(i, c):
        new_v[i, :] = jnp.zeros((H,), jnp.float32)
        return c

    lax.fori_loop(0, NPT, zrow, 0)
    pltpu.sync_copy(new_v, acc_sh.at[pl.ds(nbase, NPT)])
    pltpu.sync_copy(w1_hbm, w1_v)
    pltpu.sync_copy(bv_hbm, bv_v)
    plsc.subcore_barrier()
    sbase = sid * _EPS
    for b in range(_NB_S):
        off = sbase + b * _CHK
        pltpu.sync_copy(dst_hbm.at[pl.ds(off, _CHK)], idx_v)
        pltpu.sync_copy(msg_hbm.at[pl.ds(off, _CHK)], buf_v)
        pltpu.sync_copy(buf_v, acc_sh.at[idx_v], add=True)
    plsc.subcore_barrier()
    pltpu.sync_copy(acc_sh.at[pl.ds(nbase, NPT)], agg_v)
    pltpu.sync_copy(old_hbm.at[pl.ds(nbase, NPT)], old_v)
    pltpu.sync_copy(h0_hbm.at[pl.ds(nbase, NPT)], h0_v)

    def urow(r, c):
        rst = agg_v[r, :] + old_v[r, :] + bv_v[0, :]
        temp = ALPHA * rst + (1.0 - ALPHA) * h0_v[r, :]
        z = BETA * bv_v[1, :] + (1.0 - BETA) * temp
        for i in range(H):
            z = z + (BETA * temp[i]) * w1_v[i, :]
        new_v[r, :] = jnp.maximum(z, 0.0)
        return c

    lax.fori_loop(0, NPT, urow, 0)
    pltpu.sync_copy(new_v, tab_sh.at[pl.ds(nbase, NPT)])

    @pl.when(cid == 0)
    def _():
        pltpu.sync_copy(new_v, outn_hbm.at[pl.ds(nbase, NPT)])

    plsc.subcore_barrier()
    gbase = wid * EPW
    for b in range(_NB_G):
        off = gbase + b * _CHK
        pltpu.sync_copy(src_hbm.at[pl.ds(off, _CHK)], idx_v)
        pltpu.sync_copy(tab_sh.at[idx_v], buf_v)
        pltpu.sync_copy(buf_v, xs_hbm.at[pl.ds(off, _CHK)])


_sc_sug = pl.kernel(
    _sc_sug_body,
    out_type=(jax.ShapeDtypeStruct((NN, H), jnp.float32),
              jax.ShapeDtypeStruct((NE, H), jnp.float32)),
    mesh=_SC_MESH,
    scratch_types=[
        pltpu.VMEM((_CHK,), jnp.int32),
        pltpu.VMEM((_CHK, H), jnp.float32),
        pltpu.VMEM((NPT, H), jnp.float32),
        pltpu.VMEM((NPT, H), jnp.float32),
        pltpu.VMEM((NPT, H), jnp.float32),
        pltpu.VMEM((NPT, H), jnp.float32),
        pltpu.VMEM((H, H), jnp.float32),
        pltpu.VMEM((2, H), jnp.float32),
        pltpu.VMEM_SHARED((NN, H), jnp.float32),
        pltpu.VMEM_SHARED((NN, H), jnp.float32),
    ],
    compiler_params=pltpu.CompilerParams(use_tc_tiling_on_sc=False),
)


_NT = 2000
_ET = 8000
_EH = 16000


def _lin0_body(x_ref, w_ref, b_ref, o_ref):
    o_ref[...] = jnp.maximum(x_ref[...] @ w_ref[...] + b_ref[...], 0.0)


_lin0 = pl.pallas_call(
    _lin0_body,
    grid=(NN // _NT,),
    in_specs=[pl.BlockSpec((_NT, DIN), lambda i: (i, 0)),
              pl.BlockSpec((DIN, H), lambda i: (0, 0)),
              pl.BlockSpec((1, H), lambda i: (0, 0))],
    out_specs=pl.BlockSpec((_NT, H), lambda i: (i, 0)),
    out_shape=jax.ShapeDtypeStruct((NN, H), jnp.float32),
)


def _msg_body(xs_ref, ef_ref, we1_ref, be1_ref, we2_ref, be2_ref, r_ref,
              o_ref):
    xs_t = xs_ref[...].T
    ef_t = ef_ref[...].T
    we1_t = we1_ref[...].T
    we2_t = we2_ref[...].T
    r_t = r_ref[...].T
    be1_t = be1_ref[...].reshape(EHID, 1)
    be2_t = be2_ref[...].reshape(H * H, 1)
    mt = []
    for j in range(8):
        ef_j = ef_t[16 * j:16 * j + 16, :]
        xs_j = xs_t[16 * j:16 * j + 16, :]
        u = jnp.maximum(we1_t @ ef_j + be1_t, 0.0)
        ew = we2_t @ u + be2_t
        p = (r_t @ xs_j) * ew
        p = p[:128, :] + p[128:, :]
        p = p[:64, :] + p[64:, :]
        p = p[:32, :] + p[32:, :]
        mt.append(p[:16, :] + p[16:, :])
    o_ref[...] = jnp.concatenate(mt, axis=0).T


_msg = pl.pallas_call(
    _msg_body,
    grid=(NE // _ET,),
    in_specs=[pl.BlockSpec((_ET // 8, 128), lambda i: (i, 0)),
              pl.BlockSpec((_ET // 8, 128), lambda i: (i, 0)),
              pl.BlockSpec((H, EHID), lambda i: (0, 0)),
              pl.BlockSpec((1, EHID), lambda i: (0, 0)),
              pl.BlockSpec((EHID, H * H), lambda i: (0, 0)),
              pl.BlockSpec((1, H * H), lambda i: (0, 0)),
              pl.BlockSpec((H, H * H), lambda i: (0, 0))],
    out_specs=pl.BlockSpec((_ET // 8, 128), lambda i: (i, 0)),
    out_shape=jax.ShapeDtypeStruct((NEP, 128), jnp.float32),
)


def _upd_body(a0_ref, a1_ref, out_ref, h0_ref, bc_ref, w1_ref, b1_ref, o_ref):
    rst = a0_ref[...] + a1_ref[...] + out_ref[...] + bc_ref[...]
    temp = ALPHA * rst + (1.0 - ALPHA) * h0_ref[...]
    z = temp @ w1_ref[...] + b1_ref[...]
    o_ref[...] = jnp.maximum(BETA * z + (1.0 - BETA) * temp, 0.0)


_upd = pl.pallas_call(
    _upd_body,
    grid=(NN // _NT,),
    in_specs=[pl.BlockSpec((_NT, H), lambda i: (i, 0)),
              pl.BlockSpec((_NT, H), lambda i: (i, 0)),
              pl.BlockSpec((_NT, H), lambda i: (i, 0)),
              pl.BlockSpec((_NT, H), lambda i: (i, 0)),
              pl.BlockSpec((1, H), lambda i: (0, 0)),
              pl.BlockSpec((H, H), lambda i: (0, 0)),
              pl.BlockSpec((1, H), lambda i: (0, 0))],
    out_specs=pl.BlockSpec((_NT, H), lambda i: (i, 0)),
    out_shape=jax.ShapeDtypeStruct((NN, H), jnp.float32),
)


def _head_body(out_ref, mean_ref, var_ref, gam_ref, bet_ref, w2r_ref, w3_ref,
               b3_ref, ybn_ref, yw_ref, ysig_ref):
    ybn = ((out_ref[...] - mean_ref[...]) * lax.rsqrt(var_ref[...] + 1e-5)
           * gam_ref[...] + bet_ref[...])
    ybn_ref[...] = ybn
    yw_ref[...] = ybn * w2r_ref[...]
    ysig_ref[...] = jax.nn.sigmoid(ybn @ w3_ref[...] + b3_ref[...])


_head = pl.pallas_call(
    _head_body,
    grid=(NN // _NT,),
    in_specs=[pl.BlockSpec((_NT, H), lambda i: (i, 0)),
              pl.BlockSpec((1, H), lambda i: (0, 0)),
              pl.BlockSpec((1, H), lambda i: (0, 0)),
              pl.BlockSpec((1, H), lambda i: (0, 0)),
              pl.BlockSpec((1, H), lambda i: (0, 0)),
              pl.BlockSpec((1, H), lambda i: (0, 0)),
              pl.BlockSpec((H, 3), lambda i: (0, 0)),
              pl.BlockSpec((1, 3), lambda i: (0, 0))],
    out_specs=(pl.BlockSpec((_NT, H), lambda i: (i, 0)),
               pl.BlockSpec((_NT, H), lambda i: (i, 0)),
               pl.BlockSpec((_NT, 3), lambda i: (i, 0))),
    out_shape=(jax.ShapeDtypeStruct((NN, H), jnp.float32),
               jax.ShapeDtypeStruct((NN, H), jnp.float32),
               jax.ShapeDtypeStruct((NN, 3), jnp.float32)),
)


def _ehead_body(a_ref, b_ref, g_ref, b2_ref, o_ref):
    s = (a_ref[...] * b_ref[...]) @ g_ref[...]
    o_ref[...] = jax.nn.sigmoid(s + b2_ref[...])


_ehead = pl.pallas_call(
    _ehead_body,
    grid=(NE // _EH,),
    in_specs=[pl.BlockSpec((_EH // 8, 128), lambda i: (i, 0)),
              pl.BlockSpec((_EH // 8, 128), lambda i: (i, 0)),
              pl.BlockSpec((128, 8), lambda i: (0, 0)),
              pl.BlockSpec((1, 1), lambda i: (0, 0))],
    out_specs=pl.BlockSpec((_EH // 8, 8), lambda i: (i, 0)),
    out_shape=jax.ShapeDtypeStruct((NEP, 8), jnp.float32),
)


def _updhead_body(a0_ref, a1_ref, out_ref, h0_ref, bc_ref, w1_ref, b1_ref,
                  mean_ref, var_ref, gam_ref, bet_ref, w2r_ref, w3_ref,
                  b3_ref, ybn_ref, yw_ref, ysig_ref):
    rst = a0_ref[...] + a1_ref[...] + out_ref[...] + bc_ref[...]
    temp = ALPHA * rst + (1.0 - ALPHA) * h0_ref[...]
    z = temp @ w1_ref[...] + b1_ref[...]
    out3 = jnp.maximum(BETA * z + (1.0 - BETA) * temp, 0.0)
    ybn = ((out3 - mean_ref[...]) * lax.rsqrt(var_ref[...] + 1e-5)
           * gam_ref[...] + bet_ref[...])
    ybn_ref[...] = ybn
    yw_ref[...] = ybn * w2r_ref[...]
    ysig_ref[...] = jax.nn.sigmoid(ybn @ w3_ref[...] + b3_ref[...])


_updhead = pl.pallas_call(
    _updhead_body,
    grid=(NN // _NT,),
    in_specs=[pl.BlockSpec((_NT, H), lambda i: (i, 0)),
              pl.BlockSpec((_NT, H), lambda i: (i, 0)),
              pl.BlockSpec((_NT, H), lambda i: (i, 0)),
              pl.BlockSpec((_NT, H), lambda i: (i, 0)),
              pl.BlockSpec((1, H), lambda i: (0, 0)),
              pl.BlockSpec((H, H), lambda i: (0, 0)),
              pl.BlockSpec((1, H), lambda i: (0, 0)),
              pl.BlockSpec((1, H), lambda i: (0, 0)),
              pl.BlockSpec((1, H), lambda i: (0, 0)),
              pl.BlockSpec((1, H), lambda i: (0, 0)),
              pl.BlockSpec((1, H), lambda i: (0, 0)),
              pl.BlockSpec((1, H), lambda i: (0, 0)),
              pl.BlockSpec((H, 3), lambda i: (0, 0)),
              pl.BlockSpec((1, 3), lambda i: (0, 0))],
    out_specs=(pl.BlockSpec((_NT, H), lambda i: (i, 0)),
               pl.BlockSpec((_NT, H), lambda i: (i, 0)),
               pl.BlockSpec((_NT, 3), lambda i: (i, 0))),
    out_shape=(jax.ShapeDtypeStruct((NN, H), jnp.float32),
               jax.ShapeDtypeStruct((NN, H), jnp.float32),
               jax.ShapeDtypeStruct((NN, 3), jnp.float32)),
)


def kernel(g, n_feat, e_feat, src_list, dst_list, W0, b0, We1, be1, We2, be2,
           b_conv, W1, b1, bn_gamma, bn_beta, bn_mean, bn_var, W3, b3, W2, b2):
    src = g[0].astype(jnp.int32)
    dst = g[1].astype(jnp.int32)
    sl = src_list.astype(jnp.int32)
    dl = dst_list.astype(jnp.int32)
    cols = jnp.arange(H * H, dtype=jnp.int32) // H
    r_mat = (cols[None, :] == jnp.arange(H, dtype=jnp.int32)[:, None]
             ).astype(jnp.float32)
    lanes = jnp.arange(128, dtype=jnp.int32) // H
    g_mat = (lanes[:, None] == jnp.arange(8, dtype=jnp.int32)[None, :]
             ).astype(jnp.float32)
    ef_p = e_feat.reshape(NEP, 128)

    bvec = jnp.stack([b_conv, b1])
    out0 = _lin0(n_feat, W0, b0.reshape(1, H))
    out = out0
    for step in range(NSTEPS):
        xs = _sc_gather(out, src)
        msg_p = _msg(xs.reshape(NEP, 128), ef_p, We1, be1.reshape(1, EHID),
                     We2, be2.reshape(1, H * H), r_mat)
        msg = msg_p.reshape(NE, H)
        aggp = _sc_scatter(msg, dst)
        if step < NSTEPS - 1:
            out = _upd(aggp[0], aggp[1], out, out0, b_conv.reshape(1, H), W1,
                       b1.reshape(1, H))
    ybn, yw, ysig = _updhead(aggp[0], aggp[1], out, out0,
                             b_conv.reshape(1, H), W1, b1.reshape(1, H),
                             bn_mean.reshape(1, H), bn_var.reshape(1, H),
                             bn_gamma.reshape(1, H), bn_beta.reshape(1, H),
                             W2.reshape(1, H), W3, b3.reshape(1, 3))
    ga, gb = _sc_gather2(yw, ybn, sl, dl)
    ehop_p = _ehead(ga.reshape(NEP, 128), gb.reshape(NEP, 128), g_mat,
                    b2.reshape(1, 1))
    return (ysig, ehop_p.reshape(NE, 1))


# speedup vs baseline: 2.2334x; 1.1231x over previous
import jax
import jax.numpy as jnp
from jax import lax
from jax.experimental import pallas as pl
from jax.experimental.pallas import tpu as pltpu
from jax.experimental.pallas import tpu_sc as plsc

NN = 10000
NE = 160000
DIN = 128
H = 16
EHID = 64
NSTEPS = 3
ALPHA = 0.5
BETA = 1.0 / NSTEPS

NC = 2
NS = 16
NW = NC * NS
EPW = NE // NW
NPT = NN // NS
NEP = NE // 8

_SC_MESH = plsc.VectorSubcoreMesh(core_axis_name="c", subcore_axis_name="s")


def _sc_gather_body(table_hbm, idx_hbm, out_hbm, idx_v, rows_v, sem):
    wid = lax.axis_index("s") * NC + lax.axis_index("c")
    base = wid * EPW
    pltpu.sync_copy(idx_hbm.at[pl.ds(base, EPW)], idx_v)
    pltpu.async_copy(table_hbm.at[idx_v], rows_v, sem).wait()
    pltpu.sync_copy(rows_v, out_hbm.at[pl.ds(base, EPW)])


_sc_gather = pl.kernel(
    _sc_gather_body,
    out_type=jax.ShapeDtypeStruct((NE, H), jnp.float32),
    mesh=_SC_MESH,
    scratch_types=[
        pltpu.VMEM((EPW,), jnp.int32),
        pltpu.VMEM((EPW, H), jnp.float32),
        pltpu.SemaphoreType.DMA,
    ],
    compiler_params=pltpu.CompilerParams(use_tc_tiling_on_sc=False),
)


def _sc_gather2_body(ta_hbm, tb_hbm, ia_hbm, ib_hbm, oa_hbm, ob_hbm,
                     idx_v, rows_v, sem):
    wid = lax.axis_index("s") * NC + lax.axis_index("c")
    base = wid * EPW
    pltpu.sync_copy(ia_hbm.at[pl.ds(base, EPW)], idx_v)
    pltpu.async_copy(ta_hbm.at[idx_v], rows_v, sem).wait()
    pltpu.sync_copy(rows_v, oa_hbm.at[pl.ds(base, EPW)])
    pltpu.sync_copy(ib_hbm.at[pl.ds(base, EPW)], idx_v)
    pltpu.async_copy(tb_hbm.at[idx_v], rows_v, sem).wait()
    pltpu.sync_copy(rows_v, ob_hbm.at[pl.ds(base, EPW)])


_sc_gather2 = pl.kernel(
    _sc_gather2_body,
    out_type=(jax.ShapeDtypeStruct((NE, H), jnp.float32),
              jax.ShapeDtypeStruct((NE, H), jnp.float32)),
    mesh=_SC_MESH,
    scratch_types=[
        pltpu.VMEM((EPW,), jnp.int32),
        pltpu.VMEM((EPW, H), jnp.float32),
        pltpu.SemaphoreType.DMA,
    ],
    compiler_params=pltpu.CompilerParams(use_tc_tiling_on_sc=False),
)


def _sc_scatter_body(msg_hbm, dst_hbm, out_hbm, idx_v, msg_v, wb_v, acc_sh):
    cid = lax.axis_index("c")
    sid = lax.axis_index("s")
    wid = sid * NC + cid
    base = wid * EPW

    def zrow(i, carry):
        wb_v[i, :] = jnp.zeros((H,), jnp.float32)
        return carry

    lax.fori_loop(0, NPT, zrow, 0)
    pltpu.sync_copy(wb_v, acc_sh.at[pl.ds(sid * NPT, NPT)])
    plsc.subcore_barrier()
    pltpu.sync_copy(dst_hbm.at[pl.ds(base, EPW)], idx_v)
    pltpu.sync_copy(msg_hbm.at[pl.ds(base, EPW)], msg_v)
    pltpu.sync_copy(msg_v, acc_sh.at[idx_v], add=True)
    plsc.subcore_barrier()
    pltpu.sync_copy(acc_sh.at[pl.ds(sid * NPT, NPT)], wb_v)
    pltpu.sync_copy(wb_v, out_hbm.at[cid, pl.ds(sid * NPT, NPT)])


_sc_scatter = pl.kernel(
    _sc_scatter_body,
    out_type=jax.ShapeDtypeStruct((NC, NN, H), jnp.float32),
    mesh=_SC_MESH,
    scratch_types=[
        pltpu.VMEM((EPW,), jnp.int32),
        pltpu.VMEM((EPW, H), jnp.float32),
        pltpu.VMEM((NPT, H), jnp.float32),
        pltpu.VMEM_SHARED((NN, H), jnp.float32),
    ],
    compiler_params=pltpu.CompilerParams(use_tc_tiling_on_sc=False),
)


_CHK = 1000
_EPS = NE // NS
_NB_S = _EPS // _CHK
_NB_G = EPW // _CHK


def _sc_sug_body(msg_hbm, dst_hbm, src_hbm, old_hbm, h0_hbm, w1_hbm, bv_hbm,
                 outn_hbm, xs_hbm,
                 idx_v, buf_v, agg_v, old_v, h0_v, new_v, w1_v, bv_v,
                 acc_sh, tab_sh):
    cid = lax.axis_index("c")
    sid = lax.axis_index("s")
    wid = sid * NC + cid
    nbase = sid * NPT

    def zrow(i, c):
        new_v[i, :] = jnp.zeros((H,), jnp.float32)
        return c

    lax.fori_loop(0, NPT, zrow, 0)
    pltpu.sync_copy(new_v, acc_sh.at[pl.ds(nbase, NPT)])
    pltpu.sync_copy(w1_hbm, w1_v)
    pltpu.sync_copy(bv_hbm, bv_v)
    plsc.subcore_barrier()
    sbase = sid * _EPS
    for b in range(_NB_S):
        off = sbase + b * _CHK
        pltpu.sync_copy(dst_hbm.at[pl.ds(off, _CHK)], idx_v)
        pltpu.sync_copy(msg_hbm.at[pl.ds(off, _CHK)], buf_v)
        pltpu.sync_copy(buf_v, acc_sh.at[idx_v], add=True)
    plsc.subcore_barrier()
    pltpu.sync_copy(acc_sh.at[pl.ds(nbase, NPT)], agg_v)
    pltpu.sync_copy(old_hbm.at[pl.ds(nbase, NPT)], old_v)
    pltpu.sync_copy(h0_hbm.at[pl.ds(nbase, NPT)], h0_v)

    def urow(r, c):
        rst = agg_v[r, :] + old_v[r, :] + bv_v[0, :]
        temp = ALPHA * rst + (1.0 - ALPHA) * h0_v[r, :]
        z = BETA * bv_v[1, :] + (1.0 - BETA) * temp
        for i in range(H):
            z = z + (BETA * temp[i]) * w1_v[i, :]
        new_v[r, :] = jnp.maximum(z, 0.0)
        return c

    lax.fori_loop(0, NPT, urow, 0)
    pltpu.sync_copy(new_v, tab_sh.at[pl.ds(nbase, NPT)])

    @pl.when(cid == 0)
    def _():
        pltpu.sync_copy(new_v, outn_hbm.at[pl.ds(nbase, NPT)])

    plsc.subcore_barrier()
    gbase = wid * EPW
    for b in range(_NB_G):
        off = gbase + b * _CHK
        pltpu.sync_copy(src_hbm.at[pl.ds(off, _CHK)], idx_v)
        pltpu.sync_copy(tab_sh.at[idx_v], buf_v)
        pltpu.sync_copy(buf_v, xs_hbm.at[pl.ds(off, _CHK)])


_sc_sug = pl.kernel(
    _sc_sug_body,
    out_type=(jax.ShapeDtypeStruct((NN, H), jnp.float32),
              jax.ShapeDtypeStruct((NE, H), jnp.float32)),
    mesh=_SC_MESH,
    scratch_types=[
        pltpu.VMEM((_CHK,), jnp.int32),
        pltpu.VMEM((_CHK, H), jnp.float32),
        pltpu.VMEM((NPT, H), jnp.float32),
        pltpu.VMEM((NPT, H), jnp.float32),
        pltpu.VMEM((NPT, H), jnp.float32),
        pltpu.VMEM((NPT, H), jnp.float32),
        pltpu.VMEM((H, H), jnp.float32),
        pltpu.VMEM((2, H), jnp.float32),
        pltpu.VMEM_SHARED((NN, H), jnp.float32),
        pltpu.VMEM_SHARED((NN, H), jnp.float32),
    ],
    compiler_params=pltpu.CompilerParams(use_tc_tiling_on_sc=False),
)


_NT = 2000
_ET = 16000
_EH = 16000


def _lin0_body(x_ref, w_ref, b_ref, o_ref):
    o_ref[...] = jnp.maximum(x_ref[...] @ w_ref[...] + b_ref[...], 0.0)


_lin0 = pl.pallas_call(
    _lin0_body,
    grid=(NN // _NT,),
    in_specs=[pl.BlockSpec((_NT, DIN), lambda i: (i, 0)),
              pl.BlockSpec((DIN, H), lambda i: (0, 0)),
              pl.BlockSpec((1, H), lambda i: (0, 0))],
    out_specs=pl.BlockSpec((_NT, H), lambda i: (i, 0)),
    out_shape=jax.ShapeDtypeStruct((NN, H), jnp.float32),
)


def _msg_body(xs_ref, ef_ref, we1_ref, be1_ref, we2_ref, be2_ref, r_ref,
              o_ref):
    xs_t = xs_ref[...].T
    ef_t = ef_ref[...].T
    we1_t = we1_ref[...].T
    we2_t = we2_ref[...].T
    r_t = r_ref[...].T
    be1_t = be1_ref[...].reshape(EHID, 1)
    be2_t = be2_ref[...].reshape(H * H, 1)
    mt = []
    for j in range(8):
        ef_j = ef_t[16 * j:16 * j + 16, :]
        xs_j = xs_t[16 * j:16 * j + 16, :]
        u = jnp.maximum(we1_t @ ef_j + be1_t, 0.0)
        ew = we2_t @ u + be2_t
        p = (r_t @ xs_j) * ew
        p = p[:128, :] + p[128:, :]
        p = p[:64, :] + p[64:, :]
        p = p[:32, :] + p[32:, :]
        mt.append(p[:16, :] + p[16:, :])
    o_ref[...] = jnp.concatenate(mt, axis=0).T


_msg = pl.pallas_call(
    _msg_body,
    grid=(NE // _ET,),
    in_specs=[pl.BlockSpec((_ET // 8, 128), lambda i: (i, 0)),
              pl.BlockSpec((_ET // 8, 128), lambda i: (i, 0)),
              pl.BlockSpec((H, EHID), lambda i: (0, 0)),
              pl.BlockSpec((1, EHID), lambda i: (0, 0)),
              pl.BlockSpec((EHID, H * H), lambda i: (0, 0)),
              pl.BlockSpec((1, H * H), lambda i: (0, 0)),
              pl.BlockSpec((H, H * H), lambda i: (0, 0))],
    out_specs=pl.BlockSpec((_ET // 8, 128), lambda i: (i, 0)),
    out_shape=jax.ShapeDtypeStruct((NEP, 128), jnp.float32),
)


def _upd_body(ap_ref, out_ref, h0_ref, bc_ref, w1_ref, b1_ref, o_ref):
    rst = ap_ref[0] + ap_ref[1] + out_ref[...] + bc_ref[...]
    temp = ALPHA * rst + (1.0 - ALPHA) * h0_ref[...]
    z = temp @ w1_ref[...] + b1_ref[...]
    o_ref[...] = jnp.maximum(BETA * z + (1.0 - BETA) * temp, 0.0)


_upd = pl.pallas_call(
    _upd_body,
    grid=(NN // _NT,),
    in_specs=[pl.BlockSpec((2, _NT, H), lambda i: (0, i, 0)),
              pl.BlockSpec((_NT, H), lambda i: (i, 0)),
              pl.BlockSpec((_NT, H), lambda i: (i, 0)),
              pl.BlockSpec((1, H), lambda i: (0, 0)),
              pl.BlockSpec((H, H), lambda i: (0, 0)),
              pl.BlockSpec((1, H), lambda i: (0, 0))],
    out_specs=pl.BlockSpec((_NT, H), lambda i: (i, 0)),
    out_shape=jax.ShapeDtypeStruct((NN, H), jnp.float32),
)


def _head_body(out_ref, mean_ref, var_ref, gam_ref, bet_ref, w2r_ref, w3_ref,
               b3_ref, ybn_ref, yw_ref, ysig_ref):
    ybn = ((out_ref[...] - mean_ref[...]) * lax.rsqrt(var_ref[...] + 1e-5)
           * gam_ref[...] + bet_ref[...])
    ybn_ref[...] = ybn
    yw_ref[...] = ybn * w2r_ref[...]
    ysig_ref[...] = jax.nn.sigmoid(ybn @ w3_ref[...] + b3_ref[...])


_head = pl.pallas_call(
    _head_body,
    grid=(NN // _NT,),
    in_specs=[pl.BlockSpec((_NT, H), lambda i: (i, 0)),
              pl.BlockSpec((1, H), lambda i: (0, 0)),
              pl.BlockSpec((1, H), lambda i: (0, 0)),
              pl.BlockSpec((1, H), lambda i: (0, 0)),
              pl.BlockSpec((1, H), lambda i: (0, 0)),
              pl.BlockSpec((1, H), lambda i: (0, 0)),
              pl.BlockSpec((H, 3), lambda i: (0, 0)),
              pl.BlockSpec((1, 3), lambda i: (0, 0))],
    out_specs=(pl.BlockSpec((_NT, H), lambda i: (i, 0)),
               pl.BlockSpec((_NT, H), lambda i: (i, 0)),
               pl.BlockSpec((_NT, 3), lambda i: (i, 0))),
    out_shape=(jax.ShapeDtypeStruct((NN, H), jnp.float32),
               jax.ShapeDtypeStruct((NN, H), jnp.float32),
               jax.ShapeDtypeStruct((NN, 3), jnp.float32)),
)


def _ehead_body(a_ref, b_ref, g_ref, b2_ref, o_ref):
    s = (a_ref[...] * b_ref[...]) @ g_ref[...]
    o_ref[...] = jax.nn.sigmoid(s + b2_ref[...])


_ehead = pl.pallas_call(
    _ehead_body,
    grid=(NE // _EH,),
    in_specs=[pl.BlockSpec((_EH // 8, 128), lambda i: (i, 0)),
              pl.BlockSpec((_EH // 8, 128), lambda i: (i, 0)),
              pl.BlockSpec((128, 8), lambda i: (0, 0)),
              pl.BlockSpec((1, 1), lambda i: (0, 0))],
    out_specs=pl.BlockSpec((_EH // 8, 8), lambda i: (i, 0)),
    out_shape=jax.ShapeDtypeStruct((NEP, 8), jnp.float32),
)


def _updhead_body(ap_ref, out_ref, h0_ref, bc_ref, w1_ref, b1_ref,
                  mean_ref, var_ref, gam_ref, bet_ref, w2r_ref, w3_ref,
                  b3_ref, ybn_ref, yw_ref, ysig_ref):
    rst = ap_ref[0] + ap_ref[1] + out_ref[...] + bc_ref[...]
    temp = ALPHA * rst + (1.0 - ALPHA) * h0_ref[...]
    z = temp @ w1_ref[...] + b1_ref[...]
    out3 = jnp.maximum(BETA * z + (1.0 - BETA) * temp, 0.0)
    ybn = ((out3 - mean_ref[...]) * lax.rsqrt(var_ref[...] + 1e-5)
           * gam_ref[...] + bet_ref[...])
    ybn_ref[...] = ybn
    yw_ref[...] = ybn * w2r_ref[...]
    ysig_ref[...] = jax.nn.sigmoid(ybn @ w3_ref[...] + b3_ref[...])


_updhead = pl.pallas_call(
    _updhead_body,
    grid=(NN // _NT,),
    in_specs=[pl.BlockSpec((2, _NT, H), lambda i: (0, i, 0)),
              pl.BlockSpec((_NT, H), lambda i: (i, 0)),
              pl.BlockSpec((_NT, H), lambda i: (i, 0)),
              pl.BlockSpec((1, H), lambda i: (0, 0)),
              pl.BlockSpec((H, H), lambda i: (0, 0)),
              pl.BlockSpec((1, H), lambda i: (0, 0)),
              pl.BlockSpec((1, H), lambda i: (0, 0)),
              pl.BlockSpec((1, H), lambda i: (0, 0)),
              pl.BlockSpec((1, H), lambda i: (0, 0)),
              pl.BlockSpec((1, H), lambda i: (0, 0)),
              pl.BlockSpec((1, H), lambda i: (0, 0)),
              pl.BlockSpec((H, 3), lambda i: (0, 0)),
              pl.BlockSpec((1, 3), lambda i: (0, 0))],
    out_specs=(pl.BlockSpec((_NT, H), lambda i: (i, 0)),
               pl.BlockSpec((_NT, H), lambda i: (i, 0)),
               pl.BlockSpec((_NT, 3), lambda i: (i, 0))),
    out_shape=(jax.ShapeDtypeStruct((NN, H), jnp.float32),
               jax.ShapeDtypeStruct((NN, H), jnp.float32),
               jax.ShapeDtypeStruct((NN, 3), jnp.float32)),
)


def kernel(g, n_feat, e_feat, src_list, dst_list, W0, b0, We1, be1, We2, be2,
           b_conv, W1, b1, bn_gamma, bn_beta, bn_mean, bn_var, W3, b3, W2, b2):
    src = g[0].astype(jnp.int32)
    dst = g[1].astype(jnp.int32)
    sl = src_list.astype(jnp.int32)
    dl = dst_list.astype(jnp.int32)
    cols = jnp.arange(H * H, dtype=jnp.int32) // H
    r_mat = (cols[None, :] == jnp.arange(H, dtype=jnp.int32)[:, None]
             ).astype(jnp.float32)
    lanes = jnp.arange(128, dtype=jnp.int32) // H
    g_mat = (lanes[:, None] == jnp.arange(8, dtype=jnp.int32)[None, :]
             ).astype(jnp.float32)
    ef_p = e_feat.reshape(NEP, 128)

    bvec = jnp.stack([b_conv, b1])
    out0 = _lin0(n_feat, W0, b0.reshape(1, H))
    out = out0
    for step in range(NSTEPS):
        xs = _sc_gather(out, src)
        msg_p = _msg(xs.reshape(NEP, 128), ef_p, We1, be1.reshape(1, EHID),
                     We2, be2.reshape(1, H * H), r_mat)
        msg = msg_p.reshape(NE, H)
        aggp = _sc_scatter(msg, dst)
        if step < NSTEPS - 1:
            out = _upd(aggp, out, out0, b_conv.reshape(1, H), W1,
                       b1.reshape(1, H))
    ybn, yw, ysig = _updhead(aggp, out, out0,
                             b_conv.reshape(1, H), W1, b1.reshape(1, H),
                             bn_mean.reshape(1, H), bn_var.reshape(1, H),
                             bn_gamma.reshape(1, H), bn_beta.reshape(1, H),
                             W2.reshape(1, H), W3, b3.reshape(1, 3))
    ga, gb = _sc_gather2(yw, ybn, sl, dl)
    ehop_p = _ehead(ga.reshape(NEP, 128), gb.reshape(NEP, 128), g_mat,
                    b2.reshape(1, 1))
    return (ysig, ehop_p.reshape(NE, 1))


# speedup vs baseline: 2.2414x; 1.0036x over previous
import jax
import jax.numpy as jnp
from jax import lax
from jax.experimental import pallas as pl
from jax.experimental.pallas import tpu as pltpu
from jax.experimental.pallas import tpu_sc as plsc

NN = 10000
NE = 160000
DIN = 128
H = 16
EHID = 64
NSTEPS = 3
ALPHA = 0.5
BETA = 1.0 / NSTEPS

NC = 2
NS = 16
NW = NC * NS
EPW = NE // NW
NPT = NN // NS
NEP = NE // 8

_SC_MESH = plsc.VectorSubcoreMesh(core_axis_name="c", subcore_axis_name="s")


def _sc_gather_body(table_hbm, idx_hbm, out_hbm, idx_v, rows_v, sem):
    wid = lax.axis_index("s") * NC + lax.axis_index("c")
    base = wid * EPW
    pltpu.sync_copy(idx_hbm.at[pl.ds(base, EPW)], idx_v)
    pltpu.async_copy(table_hbm.at[idx_v], rows_v, sem).wait()
    pltpu.sync_copy(rows_v, out_hbm.at[pl.ds(base, EPW)])


_sc_gather = pl.kernel(
    _sc_gather_body,
    out_type=jax.ShapeDtypeStruct((NE, H), jnp.float32),
    mesh=_SC_MESH,
    scratch_types=[
        pltpu.VMEM((EPW,), jnp.int32),
        pltpu.VMEM((EPW, H), jnp.float32),
        pltpu.SemaphoreType.DMA,
    ],
    compiler_params=pltpu.CompilerParams(use_tc_tiling_on_sc=False),
)


def _sc_gather2_body(ta_hbm, tb_hbm, ia_hbm, ib_hbm, oa_hbm, ob_hbm,
                     idx_v, rows_v, sem):
    wid = lax.axis_index("s") * NC + lax.axis_index("c")
    base = wid * EPW
    pltpu.sync_copy(ia_hbm.at[pl.ds(base, EPW)], idx_v)
    pltpu.async_copy(ta_hbm.at[idx_v], rows_v, sem).wait()
    pltpu.sync_copy(rows_v, oa_hbm.at[pl.ds(base, EPW)])
    pltpu.sync_copy(ib_hbm.at[pl.ds(base, EPW)], idx_v)
    pltpu.async_copy(tb_hbm.at[idx_v], rows_v, sem).wait()
    pltpu.sync_copy(rows_v, ob_hbm.at[pl.ds(base, EPW)])


_sc_gather2 = pl.kernel(
    _sc_gather2_body,
    out_type=(jax.ShapeDtypeStruct((NE, H), jnp.float32),
              jax.ShapeDtypeStruct((NE, H), jnp.float32)),
    mesh=_SC_MESH,
    scratch_types=[
        pltpu.VMEM((EPW,), jnp.int32),
        pltpu.VMEM((EPW, H), jnp.float32),
        pltpu.SemaphoreType.DMA,
    ],
    compiler_params=pltpu.CompilerParams(use_tc_tiling_on_sc=False),
)


def _sc_scatter_body(msg_hbm, dst_hbm, out_hbm, idx_v, msg_v, wb_v, acc_sh):
    cid = lax.axis_index("c")
    sid = lax.axis_index("s")
    wid = sid * NC + cid
    base = wid * EPW

    def zrow(i, carry):
        wb_v[i, :] = jnp.zeros((H,), jnp.float32)
        return carry

    lax.fori_loop(0, NPT, zrow, 0)
    pltpu.sync_copy(wb_v, acc_sh.at[pl.ds(sid * NPT, NPT)])
    plsc.subcore_barrier()
    pltpu.sync_copy(dst_hbm.at[pl.ds(base, EPW)], idx_v)
    pltpu.sync_copy(msg_hbm.at[pl.ds(base, EPW)], msg_v)
    pltpu.sync_copy(msg_v, acc_sh.at[idx_v], add=True)
    plsc.subcore_barrier()
    pltpu.sync_copy(acc_sh.at[pl.ds(sid * NPT, NPT)], wb_v)
    pltpu.sync_copy(wb_v, out_hbm.at[cid, pl.ds(sid * NPT, NPT)])


_sc_scatter = pl.kernel(
    _sc_scatter_body,
    out_type=jax.ShapeDtypeStruct((NC, NN, H), jnp.float32),
    mesh=_SC_MESH,
    scratch_types=[
        pltpu.VMEM((EPW,), jnp.int32),
        pltpu.VMEM((EPW, H), jnp.float32),
        pltpu.VMEM((NPT, H), jnp.float32),
        pltpu.VMEM_SHARED((NN, H), jnp.float32),
    ],
    compiler_params=pltpu.CompilerParams(use_tc_tiling_on_sc=False),
)


_NT = 2000
_ET = 16000
_EH = 32000


def _lin0_body(x_ref, w_ref, b_ref, o_ref):
    o_ref[...] = jnp.maximum(x_ref[...] @ w_ref[...] + b_ref[...], 0.0)


_lin0 = pl.pallas_call(
    _lin0_body,
    grid=(NN // _NT,),
    in_specs=[pl.BlockSpec((_NT, DIN), lambda i: (i, 0)),
              pl.BlockSpec((DIN, H), lambda i: (0, 0)),
              pl.BlockSpec((1, H), lambda i: (0, 0))],
    out_specs=pl.BlockSpec((_NT, H), lambda i: (i, 0)),
    out_shape=jax.ShapeDtypeStruct((NN, H), jnp.float32),
)


def _msg_body(xs_ref, ef_ref, we1_ref, be1_ref, we2_ref, be2_ref, r_ref,
              o_ref):
    xs_t = xs_ref[...].T
    ef_t = ef_ref[...].T
    we1_t = we1_ref[...].T
    we2_t = we2_ref[...].T
    r_t = r_ref[...].T
    be1_t = be1_ref[...].reshape(EHID, 1)
    be2_t = be2_ref[...].reshape(H * H, 1)
    mt = []
    for j in range(8):
        ef_j = ef_t[16 * j:16 * j + 16, :]
        xs_j = xs_t[16 * j:16 * j + 16, :]
        u = jnp.maximum(we1_t @ ef_j + be1_t, 0.0)
        ew = we2_t @ u + be2_t
        p = (r_t @ xs_j) * ew
        p = p[:128, :] + p[128:, :]
        p = p[:64, :] + p[64:, :]
        p = p[:32, :] + p[32:, :]
        mt.append(p[:16, :] + p[16:, :])
    o_ref[...] = jnp.concatenate(mt, axis=0).T


_msg = pl.pallas_call(
    _msg_body,
    grid=(NE // _ET,),
    in_specs=[pl.BlockSpec((_ET // 8, 128), lambda i: (i, 0)),
              pl.BlockSpec((_ET // 8, 128), lambda i: (i, 0)),
              pl.BlockSpec((H, EHID), lambda i: (0, 0)),
              pl.BlockSpec((1, EHID), lambda i: (0, 0)),
              pl.BlockSpec((EHID, H * H), lambda i: (0, 0)),
              pl.BlockSpec((1, H * H), lambda i: (0, 0)),
              pl.BlockSpec((H, H * H), lambda i: (0, 0))],
    out_specs=pl.BlockSpec((_ET // 8, 128), lambda i: (i, 0)),
    out_shape=jax.ShapeDtypeStruct((NEP, 128), jnp.float32),
)


def _upd_body(ap_ref, out_ref, h0_ref, bc_ref, w1_ref, b1_ref, o_ref):
    rst = ap_ref[0] + ap_ref[1] + out_ref[...] + bc_ref[...]
    temp = ALPHA * rst + (1.0 - ALPHA) * h0_ref[...]
    z = temp @ w1_ref[...] + b1_ref[...]
    o_ref[...] = jnp.maximum(BETA * z + (1.0 - BETA) * temp, 0.0)


_upd = pl.pallas_call(
    _upd_body,
    grid=(NN // _NT,),
    in_specs=[pl.BlockSpec((2, _NT, H), lambda i: (0, i, 0)),
              pl.BlockSpec((_NT, H), lambda i: (i, 0)),
              pl.BlockSpec((_NT, H), lambda i: (i, 0)),
              pl.BlockSpec((1, H), lambda i: (0, 0)),
              pl.BlockSpec((H, H), lambda i: (0, 0)),
              pl.BlockSpec((1, H), lambda i: (0, 0))],
    out_specs=pl.BlockSpec((_NT, H), lambda i: (i, 0)),
    out_shape=jax.ShapeDtypeStruct((NN, H), jnp.float32),
)


def _ehead_body(a_ref, b_ref, g_ref, b2_ref, o_ref):
    s = (a_ref[...] * b_ref[...]) @ g_ref[...]
    o_ref[...] = jax.nn.sigmoid(s + b2_ref[...])


_ehead = pl.pallas_call(
    _ehead_body,
    grid=(NE // _EH,),
    in_specs=[pl.BlockSpec((_EH // 8, 128), lambda i: (i, 0)),
              pl.BlockSpec((_EH // 8, 128), lambda i: (i, 0)),
              pl.BlockSpec((128, 8), lambda i: (0, 0)),
              pl.BlockSpec((1, 1), lambda i: (0, 0))],
    out_specs=pl.BlockSpec((_EH // 8, 8), lambda i: (i, 0)),
    out_shape=jax.ShapeDtypeStruct((NEP, 8), jnp.float32),
)


def _updhead_body(ap_ref, out_ref, h0_ref, bc_ref, w1_ref, b1_ref,
                  mean_ref, var_ref, gam_ref, bet_ref, w2r_ref, w3_ref,
                  b3_ref, ybn_ref, yw_ref, ysig_ref):
    rst = ap_ref[0] + ap_ref[1] + out_ref[...] + bc_ref[...]
    temp = ALPHA * rst + (1.0 - ALPHA) * h0_ref[...]
    z = temp @ w1_ref[...] + b1_ref[...]
    out3 = jnp.maximum(BETA * z + (1.0 - BETA) * temp, 0.0)
    ybn = ((out3 - mean_ref[...]) * lax.rsqrt(var_ref[...] + 1e-5)
           * gam_ref[...] + bet_ref[...])
    ybn_ref[...] = ybn
    yw_ref[...] = ybn * w2r_ref[...]
    ysig_ref[...] = jax.nn.sigmoid(ybn @ w3_ref[...] + b3_ref[...])


_updhead = pl.pallas_call(
    _updhead_body,
    grid=(NN // _NT,),
    in_specs=[pl.BlockSpec((2, _NT, H), lambda i: (0, i, 0)),
              pl.BlockSpec((_NT, H), lambda i: (i, 0)),
              pl.BlockSpec((_NT, H), lambda i: (i, 0)),
              pl.BlockSpec((1, H), lambda i: (0, 0)),
              pl.BlockSpec((H, H), lambda i: (0, 0)),
              pl.BlockSpec((1, H), lambda i: (0, 0)),
              pl.BlockSpec((1, H), lambda i: (0, 0)),
              pl.BlockSpec((1, H), lambda i: (0, 0)),
              pl.BlockSpec((1, H), lambda i: (0, 0)),
              pl.BlockSpec((1, H), lambda i: (0, 0)),
              pl.BlockSpec((1, H), lambda i: (0, 0)),
              pl.BlockSpec((H, 3), lambda i: (0, 0)),
              pl.BlockSpec((1, 3), lambda i: (0, 0))],
    out_specs=(pl.BlockSpec((_NT, H), lambda i: (i, 0)),
               pl.BlockSpec((_NT, H), lambda i: (i, 0)),
               pl.BlockSpec((_NT, 3), lambda i: (i, 0))),
    out_shape=(jax.ShapeDtypeStruct((NN, H), jnp.float32),
               jax.ShapeDtypeStruct((NN, H), jnp.float32),
               jax.ShapeDtypeStruct((NN, 3), jnp.float32)),
)


def kernel(g, n_feat, e_feat, src_list, dst_list, W0, b0, We1, be1, We2, be2,
           b_conv, W1, b1, bn_gamma, bn_beta, bn_mean, bn_var, W3, b3, W2, b2):
    src = g[0].astype(jnp.int32)
    dst = g[1].astype(jnp.int32)
    sl = src_list.astype(jnp.int32)
    dl = dst_list.astype(jnp.int32)
    cols = jnp.arange(H * H, dtype=jnp.int32) // H
    r_mat = (cols[None, :] == jnp.arange(H, dtype=jnp.int32)[:, None]
             ).astype(jnp.float32)
    lanes = jnp.arange(128, dtype=jnp.int32) // H
    g_mat = (lanes[:, None] == jnp.arange(8, dtype=jnp.int32)[None, :]
             ).astype(jnp.float32)
    ef_p = e_feat.reshape(NEP, 128)

    out0 = _lin0(n_feat, W0, b0.reshape(1, H))
    out = out0
    for step in range(NSTEPS):
        xs = _sc_gather(out, src)
        msg_p = _msg(xs.reshape(NEP, 128), ef_p, We1, be1.reshape(1, EHID),
                     We2, be2.reshape(1, H * H), r_mat)
        msg = msg_p.reshape(NE, H)
        aggp = _sc_scatter(msg, dst)
        if step < NSTEPS - 1:
            out = _upd(aggp, out, out0, b_conv.reshape(1, H), W1,
                       b1.reshape(1, H))
    ybn, yw, ysig = _updhead(aggp, out, out0,
                             b_conv.reshape(1, H), W1, b1.reshape(1, H),
                             bn_mean.reshape(1, H), bn_var.reshape(1, H),
                             bn_gamma.reshape(1, H), bn_beta.reshape(1, H),
                             W2.reshape(1, H), W3, b3.reshape(1, 3))
    ga, gb = _sc_gather2(yw, ybn, sl, dl)
    ehop_p = _ehead(ga.reshape(NEP, 128), gb.reshape(NEP, 128), g_mat,
                    b2.reshape(1, 1))
    return (ysig, ehop_p.reshape(NE, 1))


# speedup vs baseline: 2.2513x; 1.0044x over previous
import jax
import jax.numpy as jnp
from jax import lax
from jax.experimental import pallas as pl
from jax.experimental.pallas import tpu as pltpu
from jax.experimental.pallas import tpu_sc as plsc

NN = 10000
NE = 160000
DIN = 128
H = 16
EHID = 64
NSTEPS = 3
ALPHA = 0.5
BETA = 1.0 / NSTEPS

NC = 2
NS = 16
NW = NC * NS
EPW = NE // NW
NPT = NN // NS
NEP = NE // 8

_SC_MESH = plsc.VectorSubcoreMesh(core_axis_name="c", subcore_axis_name="s")


def _sc_gather_body(table_hbm, idx_hbm, out_hbm, idx_v, rows_v, sem):
    wid = lax.axis_index("s") * NC + lax.axis_index("c")
    base = wid * EPW
    pltpu.sync_copy(idx_hbm.at[pl.ds(base, EPW)], idx_v)
    pltpu.async_copy(table_hbm.at[idx_v], rows_v, sem).wait()
    pltpu.sync_copy(rows_v, out_hbm.at[pl.ds(base, EPW)])


_sc_gather = pl.kernel(
    _sc_gather_body,
    out_type=jax.ShapeDtypeStruct((NE, H), jnp.float32),
    mesh=_SC_MESH,
    scratch_types=[
        pltpu.VMEM((EPW,), jnp.int32),
        pltpu.VMEM((EPW, H), jnp.float32),
        pltpu.SemaphoreType.DMA,
    ],
    compiler_params=pltpu.CompilerParams(use_tc_tiling_on_sc=False),
)


_G2C = 1000


def _sc_gather2_body(ta_hbm, tb_hbm, ia_hbm, ib_hbm, oa_hbm, ob_hbm,
                     ia_v, ib_v, ra_v, rb_v, sa, sb):
    wid = lax.axis_index("s") * NC + lax.axis_index("c")
    base = wid * EPW
    for c in range(EPW // _G2C):
        off = base + c * _G2C
        pltpu.sync_copy(ia_hbm.at[pl.ds(off, _G2C)], ia_v)
        da = pltpu.async_copy(ta_hbm.at[ia_v], ra_v, sa)
        pltpu.sync_copy(ib_hbm.at[pl.ds(off, _G2C)], ib_v)
        db = pltpu.async_copy(tb_hbm.at[ib_v], rb_v, sb)
        da.wait()
        pltpu.sync_copy(ra_v, oa_hbm.at[pl.ds(off, _G2C)])
        db.wait()
        pltpu.sync_copy(rb_v, ob_hbm.at[pl.ds(off, _G2C)])


_sc_gather2 = pl.kernel(
    _sc_gather2_body,
    out_type=(jax.ShapeDtypeStruct((NE, H), jnp.float32),
              jax.ShapeDtypeStruct((NE, H), jnp.float32)),
    mesh=_SC_MESH,
    scratch_types=[
        pltpu.VMEM((_G2C,), jnp.int32),
        pltpu.VMEM((_G2C,), jnp.int32),
        pltpu.VMEM((_G2C, H), jnp.float32),
        pltpu.VMEM((_G2C, H), jnp.float32),
        pltpu.SemaphoreType.DMA,
        pltpu.SemaphoreType.DMA,
    ],
    compiler_params=pltpu.CompilerParams(use_tc_tiling_on_sc=False),
)


def _sc_scatter_body(msg_hbm, dst_hbm, out_hbm, idx_v, msg_v, wb_v, acc_sh):
    cid = lax.axis_index("c")
    sid = lax.axis_index("s")
    wid = sid * NC + cid
    base = wid * EPW

    def zrow(i, carry):
        wb_v[i, :] = jnp.zeros((H,), jnp.float32)
        return carry

    lax.fori_loop(0, NPT, zrow, 0)
    pltpu.sync_copy(wb_v, acc_sh.at[pl.ds(sid * NPT, NPT)])
    plsc.subcore_barrier()
    pltpu.sync_copy(dst_hbm.at[pl.ds(base, EPW)], idx_v)
    pltpu.sync_copy(msg_hbm.at[pl.ds(base, EPW)], msg_v)
    pltpu.sync_copy(msg_v, acc_sh.at[idx_v], add=True)
    plsc.subcore_barrier()
    pltpu.sync_copy(acc_sh.at[pl.ds(sid * NPT, NPT)], wb_v)
    pltpu.sync_copy(wb_v, out_hbm.at[cid, pl.ds(sid * NPT, NPT)])


_sc_scatter = pl.kernel(
    _sc_scatter_body,
    out_type=jax.ShapeDtypeStruct((NC, NN, H), jnp.float32),
    mesh=_SC_MESH,
    scratch_types=[
        pltpu.VMEM((EPW,), jnp.int32),
        pltpu.VMEM((EPW, H), jnp.float32),
        pltpu.VMEM((NPT, H), jnp.float32),
        pltpu.VMEM_SHARED((NN, H), jnp.float32),
    ],
    compiler_params=pltpu.CompilerParams(use_tc_tiling_on_sc=False),
)


_NT = 5000
_ET = 16000
_EH = 32000


def _lin0_body(x_ref, w_ref, b_ref, o_ref):
    o_ref[...] = jnp.maximum(x_ref[...] @ w_ref[...] + b_ref[...], 0.0)


_lin0 = pl.pallas_call(
    _lin0_body,
    grid=(1,),
    in_specs=[pl.BlockSpec((NN, DIN), lambda i: (0, 0)),
              pl.BlockSpec((DIN, H), lambda i: (0, 0)),
              pl.BlockSpec((1, H), lambda i: (0, 0))],
    out_specs=pl.BlockSpec((NN, H), lambda i: (0, 0)),
    out_shape=jax.ShapeDtypeStruct((NN, H), jnp.float32),
)


def _msg_body(xs_ref, ef_ref, we1_ref, be1_ref, we2_ref, be2_ref, r_ref,
              o_ref):
    xs_t = xs_ref[...].T
    ef_t = ef_ref[...].T
    we1_t = we1_ref[...].T
    we2_t = we2_ref[...].T
    r_t = r_ref[...].T
    be1_t = be1_ref[...].reshape(EHID, 1)
    be2_t = be2_ref[...].reshape(H * H, 1)
    mt = []
    for j in range(8):
        ef_j = ef_t[16 * j:16 * j + 16, :]
        xs_j = xs_t[16 * j:16 * j + 16, :]
        u = jnp.maximum(we1_t @ ef_j + be1_t, 0.0)
        ew = we2_t @ u + be2_t
        p = (r_t @ xs_j) * ew
        p = p[:128, :] + p[128:, :]
        p = p[:64, :] + p[64:, :]
        p = p[:32, :] + p[32:, :]
        mt.append(p[:16, :] + p[16:, :])
    o_ref[...] = jnp.concatenate(mt, axis=0).T


_msg = pl.pallas_call(
    _msg_body,
    grid=(NE // _ET,),
    in_specs=[pl.BlockSpec((_ET // 8, 128), lambda i: (i, 0)),
              pl.BlockSpec((_ET // 8, 128), lambda i: (i, 0)),
              pl.BlockSpec((H, EHID), lambda i: (0, 0)),
              pl.BlockSpec((1, EHID), lambda i: (0, 0)),
              pl.BlockSpec((EHID, H * H), lambda i: (0, 0)),
              pl.BlockSpec((1, H * H), lambda i: (0, 0)),
              pl.BlockSpec((H, H * H), lambda i: (0, 0))],
    out_specs=pl.BlockSpec((_ET // 8, 128), lambda i: (i, 0)),
    out_shape=jax.ShapeDtypeStruct((NEP, 128), jnp.float32),
)


def _upd_body(ap_ref, out_ref, h0_ref, bc_ref, w1_ref, b1_ref, o_ref):
    rst = ap_ref[0] + ap_ref[1] + out_ref[...] + bc_ref[...]
    temp = ALPHA * rst + (1.0 - ALPHA) * h0_ref[...]
    z = temp @ w1_ref[...] + b1_ref[...]
    o_ref[...] = jnp.maximum(BETA * z + (1.0 - BETA) * temp, 0.0)


_upd = pl.pallas_call(
    _upd_body,
    grid=(NN // _NT,),
    in_specs=[pl.BlockSpec((2, _NT, H), lambda i: (0, i, 0)),
              pl.BlockSpec((_NT, H), lambda i: (i, 0)),
              pl.BlockSpec((_NT, H), lambda i: (i, 0)),
              pl.BlockSpec((1, H), lambda i: (0, 0)),
              pl.BlockSpec((H, H), lambda i: (0, 0)),
              pl.BlockSpec((1, H), lambda i: (0, 0))],
    out_specs=pl.BlockSpec((_NT, H), lambda i: (i, 0)),
    out_shape=jax.ShapeDtypeStruct((NN, H), jnp.float32),
)


def _ehead_body(a_ref, b_ref, g_ref, b2_ref, o_ref):
    s = (a_ref[...] * b_ref[...]) @ g_ref[...]
    o_ref[...] = jax.nn.sigmoid(s + b2_ref[...])


_ehead = pl.pallas_call(
    _ehead_body,
    grid=(NE // _EH,),
    in_specs=[pl.BlockSpec((_EH // 8, 128), lambda i: (i, 0)),
              pl.BlockSpec((_EH // 8, 128), lambda i: (i, 0)),
              pl.BlockSpec((128, 8), lambda i: (0, 0)),
              pl.BlockSpec((1, 1), lambda i: (0, 0))],
    out_specs=pl.BlockSpec((_EH // 8, 8), lambda i: (i, 0)),
    out_shape=jax.ShapeDtypeStruct((NEP, 8), jnp.float32),
)


def _updhead_body(ap_ref, out_ref, h0_ref, bc_ref, w1_ref, b1_ref,
                  mean_ref, var_ref, gam_ref, bet_ref, w2r_ref, w3_ref,
                  b3_ref, ybn_ref, yw_ref, ysig_ref):
    rst = ap_ref[0] + ap_ref[1] + out_ref[...] + bc_ref[...]
    temp = ALPHA * rst + (1.0 - ALPHA) * h0_ref[...]
    z = temp @ w1_ref[...] + b1_ref[...]
    out3 = jnp.maximum(BETA * z + (1.0 - BETA) * temp, 0.0)
    ybn = ((out3 - mean_ref[...]) * lax.rsqrt(var_ref[...] + 1e-5)
           * gam_ref[...] + bet_ref[...])
    ybn_ref[...] = ybn
    yw_ref[...] = ybn * w2r_ref[...]
    ysig_ref[...] = jax.nn.sigmoid(ybn @ w3_ref[...] + b3_ref[...])


_updhead = pl.pallas_call(
    _updhead_body,
    grid=(NN // _NT,),
    in_specs=[pl.BlockSpec((2, _NT, H), lambda i: (0, i, 0)),
              pl.BlockSpec((_NT, H), lambda i: (i, 0)),
              pl.BlockSpec((_NT, H), lambda i: (i, 0)),
              pl.BlockSpec((1, H), lambda i: (0, 0)),
              pl.BlockSpec((H, H), lambda i: (0, 0)),
              pl.BlockSpec((1, H), lambda i: (0, 0)),
              pl.BlockSpec((1, H), lambda i: (0, 0)),
              pl.BlockSpec((1, H), lambda i: (0, 0)),
              pl.BlockSpec((1, H), lambda i: (0, 0)),
              pl.BlockSpec((1, H), lambda i: (0, 0)),
              pl.BlockSpec((1, H), lambda i: (0, 0)),
              pl.BlockSpec((H, 3), lambda i: (0, 0)),
              pl.BlockSpec((1, 3), lambda i: (0, 0))],
    out_specs=(pl.BlockSpec((_NT, H), lambda i: (i, 0)),
               pl.BlockSpec((_NT, H), lambda i: (i, 0)),
               pl.BlockSpec((_NT, 3), lambda i: (i, 0))),
    out_shape=(jax.ShapeDtypeStruct((NN, H), jnp.float32),
               jax.ShapeDtypeStruct((NN, H), jnp.float32),
               jax.ShapeDtypeStruct((NN, 3), jnp.float32)),
)


def kernel(g, n_feat, e_feat, src_list, dst_list, W0, b0, We1, be1, We2, be2,
           b_conv, W1, b1, bn_gamma, bn_beta, bn_mean, bn_var, W3, b3, W2, b2):
    src = g[0].astype(jnp.int32)
    dst = g[1].astype(jnp.int32)
    sl = src_list.astype(jnp.int32)
    dl = dst_list.astype(jnp.int32)
    cols = jnp.arange(H * H, dtype=jnp.int32) // H
    r_mat = (cols[None, :] == jnp.arange(H, dtype=jnp.int32)[:, None]
             ).astype(jnp.float32)
    lanes = jnp.arange(128, dtype=jnp.int32) // H
    g_mat = (lanes[:, None] == jnp.arange(8, dtype=jnp.int32)[None, :]
             ).astype(jnp.float32)
    ef_p = e_feat.reshape(NEP, 128)

    out0 = _lin0(n_feat, W0, b0.reshape(1, H))
    out = out0
    for step in range(NSTEPS):
        xs = _sc_gather(out, src)
        msg_p = _msg(xs.reshape(NEP, 128), ef_p, We1, be1.reshape(1, EHID),
                     We2, be2.reshape(1, H * H), r_mat)
        msg = msg_p.reshape(NE, H)
        aggp = _sc_scatter(msg, dst)
        if step < NSTEPS - 1:
            out = _upd(aggp, out, out0, b_conv.reshape(1, H), W1,
                       b1.reshape(1, H))
    ybn, yw, ysig = _updhead(aggp, out, out0,
                             b_conv.reshape(1, H), W1, b1.reshape(1, H),
                             bn_mean.reshape(1, H), bn_var.reshape(1, H),
                             bn_gamma.reshape(1, H), bn_beta.reshape(1, H),
                             W2.reshape(1, H), W3, b3.reshape(1, 3))
    ga, gb = _sc_gather2(yw, ybn, sl, dl)
    ehop_p = _ehead(ga.reshape(NEP, 128), gb.reshape(NEP, 128), g_mat,
                    b2.reshape(1, 1))
    return (ysig, ehop_p.reshape(NE, 1))
